# Initial kernel scaffold; baseline (speedup 1.0000x reference)
import jax
import jax.numpy as jnp
from jax.experimental import pallas as pl


def kernel(x, edge_index, edge_attr, params):
    raise NotImplementedError("write your pallas kernel here")


# speedup vs baseline: 6.2618x; 6.2618x over previous
import functools

import numpy as np
import jax
import jax.numpy as jnp
from jax import lax
from jax.experimental import pallas as pl
from jax.experimental.pallas import tpu as pltpu
from jax.experimental.pallas import tpu_sc as plsc

N_NODES = 10000
N_EDGES = 320000
EP = 322560
PAD_E = EP - N_EDGES
P1 = 12
P2 = 8
R1R = EP // P1
R2R = EP // P2
NC, NS = 2, 16
NW = NC * NS
EWK = EP // NW
CH = 1008
NCH = EWK // CH
SB = 112
NACC = 10240
NROW_T = NACC // NS

EBLK1 = 1920
NBLK1 = R1R // EBLK1
EBLK2 = 2016
NBLK2 = R2R // EBLK2
NBLK_N = 5
NODE_BLK = N_NODES // NBLK_N


def _np_sel(n_in, n_out, pairs):
    s = np.zeros((n_in, n_out), np.float32)
    for i, j in pairs:
        s[i, j] = 1.0
    return s

S_E1 = _np_sel(48, 128, [(4 * p + c, 10 * p + c) for p in range(P1) for c in range(4)])
S_W1 = _np_sel(12, 128, [(p, 10 * p) for p in range(P1)])
S_EA1 = _np_sel(48, 128, [(4 * p + c, 10 * p + 1 + c) for p in range(P1) for c in range(4)])
S_E2 = _np_sel(40, 128, [(5 * p + c, 16 * p + c) for p in range(P2) for c in range(5)])


def _relu(v):
    return jnp.maximum(v, 0.0)


def _bd(w, p):
    return jnp.kron(jnp.eye(p, dtype=jnp.float32), w)


def _bdb(b, p):
    return jnp.tile(b, p)[None, :]


def _rs(blk, d):
    return pl.BlockSpec((blk, d), lambda i: (i, 0))


def _fs(a):
    return pl.BlockSpec(a.shape, lambda i: (0,) * a.ndim)


def _mlp2_body(x, w1, b1, w2, b2, o):
    t = _relu(x[...] @ w1[...] + b1[...])
    o[...] = _relu(t @ w2[...] + b2[...])


def _ec_edge_body(m, ear, we1, be1, we2, be2, w1m, w1e, b1, w2, b2, w3, b3,
                  w4, b4, v1, c1, v2, c2, v3, c3, v4, c4,
                  se1, sw1, sea1, epad_o, eaw_o, w12_o):
    ea0 = _relu(ear[...] @ we1[...] + be1[...])
    ea0 = _relu(ea0 @ we2[...] + be2[...])
    t = _relu(m[...] @ w1m[...] + ea0 @ w1e[...] + b1[...])
    t = _relu(t @ w2[...] + b2[...])
    t = _relu(t @ w3[...] + b3[...])
    e48 = t @ w4[...] + b4[...]
    ea1 = ea0 + e48
    epad_o[...] = e48 @ se1[...]
    u = _relu(ea1 @ v1[...] + c1[...])
    u = _relu(u @ v2[...] + c2[...])
    u = _relu(u @ v3[...] + c3[...])
    w12 = jax.nn.sigmoid(u @ v4[...] + c4[...])
    w12_o[...] = w12
    eaw_o[...] = w12 @ sw1[...] + ea1 @ sea1[...]


def _hc_edge_body(m, w1m, b1, w2, b2, w3, b3, w4, b4, se2, epad_o):
    t = _relu(m[...] @ w1m[...] + b1[...])
    t = _relu(t @ w2[...] + b2[...])
    t = _relu(t @ w3[...] + b3[...])
    e40 = t @ w4[...] + b4[...]
    epad_o[...] = e40 @ se2[...]


def _node_o_body(h, a0, a1, wh, wa, b1, w2, b2, w3, b3, w4, b4, o):
    agg = a0[...] + a1[...]
    t = _relu(h[...] @ wh[...] + agg @ wa[...] + b1[...])
    t = _relu(t @ w2[...] + b2[...])
    t = _relu(t @ w3[...] + b3[...])
    o[...] = h[...] + (t @ w4[...] + b4[...])


def _node_final_body(h, a0, a1, wh, wa, b1, w2, b2, w3, b3, w4, b4,
                     bw1, bb1, bw2, bb2, bw3, bb3, bw4, bb4,
                     xw1, xb1, xw2, xb2, xw3, xb3, xw4, xb4,
                     hout_o, beta_o):
    agg = a0[...] + a1[...]
    t = _relu(h[...] @ wh[...] + agg @ wa[...] + b1[...])
    t = _relu(t @ w2[...] + b2[...])
    t = _relu(t @ w3[...] + b3[...])
    h2 = h[...] + (t @ w4[...] + b4[...])
    u = _relu(h2 @ bw1[...] + bb1[...])
    u = _relu(u @ bw2[...] + bb2[...])
    u = _relu(u @ bw3[...] + bb3[...])
    beta_o[...] = jax.nn.sigmoid(u @ bw4[...] + bb4[...])
    v = _relu(h2 @ xw1[...] + xb1[...])
    v = _relu(v @ xw2[...] + xb2[...])
    v = _relu(v @ xw3[...] + xb3[...])
    hout_o[...] = v @ xw4[...] + xb4[...]


def _tc_rows(body, n_out, out_shapes, blk, nblk, row_args, full_args):
    in_specs = [_rs(blk, a.shape[1]) for a in row_args] + [_fs(a) for a in full_args]
    out_specs = tuple(_rs(blk, s[1]) for s in out_shapes)
    out_shape = tuple(jax.ShapeDtypeStruct(s, jnp.float32) for s in out_shapes)
    if n_out == 1:
        out_specs, out_shape = out_specs[0], out_shape[0]
    return pl.pallas_call(
        body, grid=(nblk,), in_specs=in_specs, out_specs=out_specs,
        out_shape=out_shape,
    )(*row_args, *full_args)


def _make_gather(p, stride, with_eaw):
    rpc = CH // p
    mlen = rpc * 128
    mesh = plsc.VectorSubcoreMesh(core_axis_name="c", subcore_axis_name="s",
                                  num_cores=NC, num_subcores=NS)
    scratch = [
        pltpu.VMEM((N_NODES * 5,), jnp.float32),
        pltpu.VMEM((CH,), jnp.int32),
        pltpu.VMEM((CH,), jnp.int32),
        pltpu.VMEM((mlen,), jnp.float32),
    ]
    if with_eaw:
        scratch.append(pltpu.VMEM(((CH // P1) * 128,), jnp.float32))

    def body(h_hbm, src_hbm, dst_hbm, *rest):
        if with_eaw:
            eaw_hbm, out_hbm, tab_v, si_v, di_v, m_v, ew_v = rest
        else:
            out_hbm, tab_v, si_v, di_v, m_v = rest
        wid = lax.axis_index("s") * NC + lax.axis_index("c")
        pltpu.sync_copy(h_hbm, tab_v)
        lanes = lax.iota(jnp.int32, 16)
        for kk in range(NCH):
            base = wid * EWK + kk * CH
            pltpu.sync_copy(src_hbm.at[pl.ds(base, CH)], si_v)
            pltpu.sync_copy(dst_hbm.at[pl.ds(base, CH)], di_v)
            if with_eaw:
                pltpu.sync_copy(
                    eaw_hbm.at[pl.ds((base // P1) * 128, (CH // P1) * 128)], ew_v)

            def grp(i, _):
                q = i * 16 + lanes
                d5 = di_v[pl.ds(i * 16, 16)] * 5
                s5 = si_v[pl.ds(i * 16, 16)] * 5
                moff = (q // p) * 128 + (q % p) * stride
                for c in range(5):
                    vd = plsc.load_gather(tab_v, [d5 + c])
                    plsc.store_scatter(m_v, [moff + c], vd)
                    vs = plsc.load_gather(tab_v, [s5 + c])
                    plsc.store_scatter(m_v, [moff + 5 + c], vs)
                if with_eaw:
                    eoff = (q // P1) * 128 + (q % P1) * 10
                    for c in range(5):
                        ve = plsc.load_gather(ew_v, [eoff + c])
                        plsc.store_scatter(m_v, [moff + 10 + c], ve)
                return 0

            lax.fori_loop(0, CH // 16, grp, 0)

            zpad = 128 - p * stride
            zero16 = jnp.zeros((16,), jnp.float32)

            def zfill(i, _):
                v = i * 16 + lanes
                addr = (v // zpad) * 128 + p * stride + (v % zpad)
                plsc.store_scatter(m_v, [addr], zero16)
                return 0

            lax.fori_loop(0, rpc * zpad // 16, zfill, 0)
            pltpu.sync_copy(m_v, out_hbm.at[pl.ds((base // p) * 128, mlen)])

    n_rows = EP // p
    in_types = [None]
    return functools.partial(
        pl.kernel, mesh=mesh,
        compiler_params=pltpu.CompilerParams(needs_layout_passes=False),
        out_type=jax.ShapeDtypeStruct((n_rows * 128,), jnp.float32),
        scratch_types=scratch,
    )(body)


def _make_scatter(p, stride):
    rpc = CH // p
    elen = rpc * 128
    mesh = plsc.VectorSubcoreMesh(core_axis_name="c", subcore_axis_name="s",
                                  num_cores=NC, num_subcores=NS)

    @functools.partial(
        pl.kernel, mesh=mesh,
        compiler_params=pltpu.CompilerParams(needs_layout_passes=False,
                                             use_tc_tiling_on_sc=False),
        out_type=jax.ShapeDtypeStruct((NC, NACC, 8), jnp.float32),
        scratch_types=[
            pltpu.VMEM((elen,), jnp.float32),
            pltpu.VMEM((CH,), jnp.int32),
            pltpu.VMEM((SB,), jnp.int32),
            pltpu.VMEM((SB, 8), jnp.float32),
            pltpu.VMEM((NROW_T, 8), jnp.float32),
            pltpu.VMEM_SHARED((NACC, 8), jnp.float32),
        ],
    )
    def body(e_hbm, d_hbm, out_hbm, e_v, di_v, i_v, s_v, zo_v, acc_sh):
        cid = lax.axis_index("c")
        sid = lax.axis_index("s")
        wid = sid * NC + cid
        lanes = lax.iota(jnp.int32, 16)
        zero16 = jnp.zeros((16,), jnp.float32)

        def zf(i, _):
            v = i * 16 + lanes
            plsc.store_scatter(zo_v, [v // 8, v % 8], zero16)
            return 0

        lax.fori_loop(0, NROW_T * 8 // 16, zf, 0)
        pltpu.sync_copy(zo_v, acc_sh.at[pl.ds(sid * NROW_T, NROW_T)])
        plsc.subcore_barrier()

        for kk in range(NCH):
            base = wid * EWK + kk * CH
            pltpu.sync_copy(e_hbm.at[pl.ds((base // p) * 128, elen)], e_v)
            pltpu.sync_copy(d_hbm.at[pl.ds(base, CH)], di_v)

            def batch(b, _):
                def fill(g, _):
                    t0 = b * SB + g * 16
                    q = t0 + lanes
                    rows = g * 16 + lanes
                    dv = di_v[pl.ds(t0, 16)]
                    eoff = (q // p) * 128 + (q % p) * stride
                    for c in range(8):
                        v = plsc.load_gather(e_v, [eoff + c])
                        plsc.store_scatter(
                            s_v, [rows, jnp.full((16,), c, jnp.int32)], v)
                    plsc.store_scatter(i_v, [rows], dv)
                    return 0

                lax.fori_loop(0, SB // 16, fill, 0)
                pltpu.sync_copy(s_v, acc_sh.at[i_v], add=True)
                return 0

            lax.fori_loop(0, CH // SB, batch, 0)

        plsc.subcore_barrier()
        pltpu.sync_copy(acc_sh.at[pl.ds(sid * NROW_T, NROW_T)], zo_v)
        pltpu.sync_copy(zo_v, out_hbm.at[cid, pl.ds(sid * NROW_T, NROW_T)])

    return body


@functools.cache
def _sc_kernels():
    return (_make_gather(P1, 10, False), _make_gather(P2, 15, True),
            _make_scatter(P1, 10), _make_scatter(P2, 16))


def kernel(x, edge_index, edge_attr, params):
    src = edge_index[0]
    dst = edge_index[1]
    zpad_i = jnp.zeros((PAD_E,), jnp.int32)
    src_p = jnp.concatenate([src, zpad_i])
    dst_g = jnp.concatenate([dst, zpad_i])
    dst_s = jnp.concatenate([dst, 10000 + (jnp.arange(PAD_E, dtype=jnp.int32) % 240)])
    ear = jnp.concatenate([edge_attr,
                           jnp.zeros((PAD_E, 4), jnp.float32)]).reshape(R1R, 48)

    pn = params['node_encoder']
    pe = params['edge_encoder']
    ec = params['ec_layers'][0]
    hc = params['hc_layers'][0]
    pw, pb, px = params['W'], params['B'], params['X']

    def b2(v):
        return v[None, :]

    def padr(w, rows):
        return jnp.concatenate([w, jnp.zeros((rows - w.shape[0], w.shape[1]),
                                             jnp.float32)], axis=0)

    g1, g2, s1, s2 = _sc_kernels()

    h0 = _tc_rows(_mlp2_body, 1, [(N_NODES, 5)], NODE_BLK, NBLK_N,
                  [x], [pn[0][0], b2(pn[0][1]), pn[1][0], b2(pn[1][1])])

    m1 = g1(h0.reshape(-1), src_p, dst_g).reshape(R1R, 128)

    r1 = ec['R1']
    epad1, eaw, w12 = _tc_rows(
        _ec_edge_body, 3, [(R1R, 128), (R1R, 128), (R1R, 12)], EBLK1, NBLK1,
        [m1, ear],
        [_bd(pe[0][0], P1), _bdb(pe[0][1], P1), _bd(pe[1][0], P1), _bdb(pe[1][1], P1),
         padr(_bd(r1[0][0][0:10], P1), 128), _bd(r1[0][0][10:14], P1), _bdb(r1[0][1], P1),
         _bd(r1[1][0], P1), _bdb(r1[1][1], P1), _bd(r1[2][0], P1), _bdb(r1[2][1], P1),
         _bd(r1[3][0], P1), _bdb(r1[3][1], P1),
         _bd(pw[0][0], P1), _bdb(pw[0][1], P1), _bd(pw[1][0], P1), _bdb(pw[1][1], P1),
         _bd(pw[2][0], P1), _bdb(pw[2][1], P1), _bd(pw[3][0], P1), _bdb(pw[3][1], P1),
         S_E1, S_W1, S_EA1])

    agg1 = s1(epad1.reshape(-1), dst_s)[:, :N_NODES]

    o1 = ec['O']
    wo_a8 = padr(o1[0][0][5:9], 8)
    h1 = _tc_rows(_node_o_body, 1, [(N_NODES, 5)], NODE_BLK, NBLK_N,
                  [h0, agg1[0], agg1[1]],
                  [o1[0][0][0:5], wo_a8, b2(o1[0][1]),
                   o1[1][0], b2(o1[1][1]), o1[2][0], b2(o1[2][1]),
                   o1[3][0], b2(o1[3][1])])

    m2 = g2(h1.reshape(-1), src_p, dst_g, eaw.reshape(-1)).reshape(R2R, 128)

    r2 = hc['R1']
    epad2 = _tc_rows(
        _hc_edge_body, 1, [(R2R, 128)], EBLK2, NBLK2,
        [m2],
        [padr(_bd(r2[0][0], P2), 128), _bdb(r2[0][1], P2),
         _bd(r2[1][0], P2), _bdb(r2[1][1], P2),
         _bd(r2[2][0], P2), _bdb(r2[2][1], P2),
         _bd(r2[3][0], P2), _bdb(r2[3][1], P2), S_E2])

    agg2 = s2(epad2.reshape(-1), dst_s)[:, :N_NODES]

    o2 = hc['O']
    wo2_a8 = padr(o2[0][0][5:10], 8)
    h_out, beta = _tc_rows(
        _node_final_body, 2, [(N_NODES, 2), (N_NODES, 1)], NODE_BLK, NBLK_N,
        [h1, agg2[0], agg2[1]],
        [o2[0][0][0:5], wo2_a8, b2(o2[0][1]),
         o2[1][0], b2(o2[1][1]), o2[2][0], b2(o2[2][1]),
         o2[3][0], b2(o2[3][1]),
         pb[0][0], b2(pb[0][1]), pb[1][0], b2(pb[1][1]),
         pb[2][0], b2(pb[2][1]), pb[3][0], b2(pb[3][1]),
         px[0][0], b2(px[0][1]), px[1][0], b2(px[1][1]),
         px[2][0], b2(px[2][1]), px[3][0], b2(px[3][1])])

    ew = w12.reshape(-1)[:N_EDGES].reshape(N_EDGES, 1)
    return (ew, h_out, beta)


# speedup vs baseline: 7.2964x; 1.1652x over previous
import functools

import numpy as np
import jax
import jax.numpy as jnp
from jax import lax
from jax.experimental import pallas as pl
from jax.experimental.pallas import tpu as pltpu
from jax.experimental.pallas import tpu_sc as plsc

N_NODES = 10000
N_EDGES = 320000
EP = 322560
PAD_E = EP - N_EDGES
P1 = 12
P2 = 8
R1R = EP // P1
R2R = EP // P2
NC, NS = 2, 16
NW = NC * NS
EWK = EP // NW
CH1 = 2016
CH2 = 1008
CHS = 2016
SB = 112
NBAT = CHS // SB
NACC = 10240
NROW_T = NACC // NS

EBLK1 = 1920
NBLK1 = R1R // EBLK1
EBLK2 = 2016
NBLK2 = R2R // EBLK2
NBLK_N = 5
NODE_BLK = N_NODES // NBLK_N


def _np_sel(n_in, n_out, pairs):
    s = np.zeros((n_in, n_out), np.float32)
    for i, j in pairs:
        s[i, j] = 1.0
    return s

S_E1 = _np_sel(48, 128, [(4 * p + c, 10 * p + c) for p in range(P1) for c in range(4)])
S_W1 = _np_sel(12, 128, [(p, 10 * p) for p in range(P1)])
S_EA1 = _np_sel(48, 128, [(4 * p + c, 10 * p + 1 + c) for p in range(P1) for c in range(4)])
S_E2 = _np_sel(40, 128, [(5 * p + c, 16 * p + c) for p in range(P2) for c in range(5)])


def _relu(v):
    return jnp.maximum(v, 0.0)


def _bd(w, p):
    return jnp.kron(jnp.eye(p, dtype=jnp.float32), w)


def _bdb(b, p):
    return jnp.tile(b, p)[None, :]


def _rs(blk, d):
    return pl.BlockSpec((blk, d), lambda i: (i, 0))


def _fs(a):
    return pl.BlockSpec(a.shape, lambda i: (0,) * a.ndim)


def _mlp2_body(x, w1, b1, w2, b2, o):
    t = _relu(x[...] @ w1[...] + b1[...])
    o[...] = _relu(t @ w2[...] + b2[...])


def _ec_edge_body(m, ear, we1, be1, we2, be2, w1m, w1e, b1, w2, b2, w3, b3,
                  w4, b4, v1, c1, v2, c2, v3, c3, v4, c4,
                  se1, sw1, sea1, epad_o, eaw_o, w12_o):
    ea0 = _relu(ear[...] @ we1[...] + be1[...])
    ea0 = _relu(ea0 @ we2[...] + be2[...])
    t = _relu(m[...] @ w1m[...] + ea0 @ w1e[...] + b1[...])
    t = _relu(t @ w2[...] + b2[...])
    t = _relu(t @ w3[...] + b3[...])
    e48 = t @ w4[...] + b4[...]
    ea1 = ea0 + e48
    epad_o[...] = e48 @ se1[...]
    u = _relu(ea1 @ v1[...] + c1[...])
    u = _relu(u @ v2[...] + c2[...])
    u = _relu(u @ v3[...] + c3[...])
    w12 = jax.nn.sigmoid(u @ v4[...] + c4[...])
    w12_o[...] = w12
    eaw_o[...] = w12 @ sw1[...] + ea1 @ sea1[...]


def _hc_edge_body(m, w1m, b1, w2, b2, w3, b3, w4, b4, se2, epad_o):
    t = _relu(m[...] @ w1m[...] + b1[...])
    t = _relu(t @ w2[...] + b2[...])
    t = _relu(t @ w3[...] + b3[...])
    e40 = t @ w4[...] + b4[...]
    epad_o[...] = e40 @ se2[...]


def _node_o_body(h, a0, a1, wh, wa, b1, w2, b2, w3, b3, w4, b4, o):
    agg = a0[...] + a1[...]
    t = _relu(h[...] @ wh[...] + agg @ wa[...] + b1[...])
    t = _relu(t @ w2[...] + b2[...])
    t = _relu(t @ w3[...] + b3[...])
    o[...] = h[...] + (t @ w4[...] + b4[...])


def _node_final_body(h, a0, a1, wh, wa, b1, w2, b2, w3, b3, w4, b4,
                     bw1, bb1, bw2, bb2, bw3, bb3, bw4, bb4,
                     xw1, xb1, xw2, xb2, xw3, xb3, xw4, xb4,
                     hout_o, beta_o):
    agg = a0[...] + a1[...]
    t = _relu(h[...] @ wh[...] + agg @ wa[...] + b1[...])
    t = _relu(t @ w2[...] + b2[...])
    t = _relu(t @ w3[...] + b3[...])
    h2 = h[...] + (t @ w4[...] + b4[...])
    u = _relu(h2 @ bw1[...] + bb1[...])
    u = _relu(u @ bw2[...] + bb2[...])
    u = _relu(u @ bw3[...] + bb3[...])
    beta_o[...] = jax.nn.sigmoid(u @ bw4[...] + bb4[...])
    v = _relu(h2 @ xw1[...] + xb1[...])
    v = _relu(v @ xw2[...] + xb2[...])
    v = _relu(v @ xw3[...] + xb3[...])
    hout_o[...] = v @ xw4[...] + xb4[...]


def _tc_rows(body, n_out, out_shapes, blk, nblk, row_args, full_args):
    in_specs = [_rs(blk, a.shape[1]) for a in row_args] + [_fs(a) for a in full_args]
    out_specs = tuple(_rs(blk, s[1]) for s in out_shapes)
    out_shape = tuple(jax.ShapeDtypeStruct(s, jnp.float32) for s in out_shapes)
    if n_out == 1:
        out_specs, out_shape = out_specs[0], out_shape[0]
    return pl.pallas_call(
        body, grid=(nblk,), in_specs=in_specs, out_specs=out_specs,
        out_shape=out_shape,
    )(*row_args, *full_args)


_N_PREP_IN = 30


def _prep_body(*refs):
    ins = refs[:_N_PREP_IN]
    outs = refs[_N_PREP_IN:]

    def bd(w, p, pad_rows=0):
        a, b = w.shape
        t = jnp.concatenate([w] * p, axis=0)
        t = jnp.concatenate([t] * p, axis=1)
        ri = lax.broadcasted_iota(jnp.int32, t.shape, 0) // a
        ci = lax.broadcasted_iota(jnp.int32, t.shape, 1) // b
        t = jnp.where(ri == ci, t, 0.0)
        if pad_rows:
            t = jnp.concatenate(
                [t, jnp.zeros((pad_rows, t.shape[1]), jnp.float32)], axis=0)
        return t

    def tl(b, p):
        return jnp.concatenate([b] * p, axis=1)

    (we1, be1, we2, be2, a1, ab1, a2, ab2, a3, ab3, a4, ab4,
     v1, vb1, v2, vb2, v3, vb3, v4, vb4,
     g1, gb1, g2, gb2, g3, gb3, g4, gb4, o1w, o2w) = [r[...] for r in ins]

    vals = [
        bd(we1, P1), tl(be1, P1), bd(we2, P1), tl(be2, P1),
        bd(a1[0:10], P1, 8), bd(a1[10:14], P1), tl(ab1, P1),
        bd(a2, P1), tl(ab2, P1), bd(a3, P1), tl(ab3, P1),
        bd(a4, P1), tl(ab4, P1),
        bd(v1, P1), tl(vb1, P1), bd(v2, P1), tl(vb2, P1),
        bd(v3, P1), tl(vb3, P1), bd(v4, P1), tl(vb4, P1),
        bd(g1, P2, 8), tl(gb1, P2), bd(g2, P2), tl(gb2, P2),
        bd(g3, P2), tl(gb3, P2), bd(g4, P2), tl(gb4, P2),
        o1w[0:5],
        jnp.concatenate([o1w[5:9], jnp.zeros((4, 40), jnp.float32)], axis=0),
        o2w[0:5],
        jnp.concatenate([o2w[5:10], jnp.zeros((3, 40), jnp.float32)], axis=0),
    ]
    for o, v in zip(outs, vals, strict=True):
        o[...] = v


def _prep_weights(ins):
    out_shapes = [
        (48, 480), (1, 480), (480, 48), (1, 48),
        (128, 480), (48, 480), (1, 480),
        (480, 480), (1, 480), (480, 480), (1, 480),
        (480, 48), (1, 48),
        (48, 480), (1, 480), (480, 480), (1, 480),
        (480, 480), (1, 480), (480, 12), (1, 12),
        (128, 320), (1, 320), (320, 320), (1, 320),
        (320, 320), (1, 320), (320, 40), (1, 40),
        (5, 40), (8, 40), (5, 40), (8, 40),
    ]
    return pl.pallas_call(
        _prep_body, grid=(1,),
        in_specs=[_fs(a) for a in ins],
        out_specs=tuple(pl.BlockSpec(s, lambda i: (0, 0)) for s in out_shapes),
        out_shape=tuple(jax.ShapeDtypeStruct(s, jnp.float32) for s in out_shapes),
    )(*ins)


def _make_gather(p, stride, with_eaw, ch):
    rpc = ch // p
    mlen = rpc * 128
    nch = EWK // ch
    ewlen = (ch // P1) * 128
    mesh = plsc.VectorSubcoreMesh(core_axis_name="c", subcore_axis_name="s",
                                  num_cores=NC, num_subcores=NS)
    scratch = [
        pltpu.VMEM((N_NODES * 5,), jnp.float32),
        pltpu.VMEM((ch,), jnp.int32), pltpu.VMEM((ch,), jnp.int32),
        pltpu.VMEM((ch,), jnp.int32), pltpu.VMEM((ch,), jnp.int32),
        pltpu.VMEM((mlen,), jnp.float32), pltpu.VMEM((mlen,), jnp.float32),
        pltpu.SemaphoreType.DMA, pltpu.SemaphoreType.DMA,
        pltpu.SemaphoreType.DMA, pltpu.SemaphoreType.DMA,
    ]
    if with_eaw:
        scratch += [pltpu.VMEM((ewlen,), jnp.float32),
                    pltpu.VMEM((ewlen,), jnp.float32)]

    def body(h_hbm, src_hbm, dst_hbm, *rest):
        if with_eaw:
            (eaw_hbm, out_hbm, tab_v, si0, si1, di0, di1, m0, m1,
             sin0, sin1, so0, so1, ew0, ew1) = rest
            ew = [ew0, ew1]
        else:
            (out_hbm, tab_v, si0, si1, di0, di1, m0, m1,
             sin0, sin1, so0, so1) = rest
            ew = None
        si, di, m = [si0, si1], [di0, di1], [m0, m1]
        sin, so = [sin0, sin1], [so0, so1]
        wid = lax.axis_index("s") * NC + lax.axis_index("c")
        lanes = lax.iota(jnp.int32, 16)
        zero16 = jnp.zeros((16,), jnp.float32)

        def in_copies(kk, s):
            base = wid * EWK + kk * ch
            yield src_hbm.at[pl.ds(base, ch)], si[s], sin[s]
            yield dst_hbm.at[pl.ds(base, ch)], di[s], sin[s]
            if with_eaw:
                yield eaw_hbm.at[pl.ds((base // P1) * 128, ewlen)], ew[s], sin[s]

        def out_copy(kk, s):
            base = wid * EWK + kk * ch
            return m[s], out_hbm.at[pl.ds((base // p) * 128, mlen)], so[s]

        for t in in_copies(0, 0):
            pltpu.async_copy(*t)
        pltpu.sync_copy(h_hbm, tab_v)

        zpad = 128 - p * stride

        def zfill(mv):
            def zf(i, _):
                v = i * 16 + lanes
                plsc.store_scatter(mv, [(v // zpad) * 128 + p * stride + (v % zpad)],
                                   zero16)
                return 0
            lax.fori_loop(0, rpc * zpad // 16, zf, 0)

        zfill(m0)
        zfill(m1)

        for kk in range(nch):
            s = kk % 2
            if kk + 1 < nch:
                for t in in_copies(kk + 1, 1 - s):
                    pltpu.async_copy(*t)
            for t in in_copies(kk, s):
                pltpu.make_async_copy(*t).wait()
            if kk >= 2:
                pltpu.make_async_copy(*out_copy(kk - 2, s)).wait()

            def grp(i, _):
                q = i * 16 + lanes
                d5 = di[s][pl.ds(i * 16, 16)] * 5
                s5 = si[s][pl.ds(i * 16, 16)] * 5
                moff = (q // p) * 128 + (q % p) * stride
                for c in range(5):
                    vd = plsc.load_gather(tab_v, [d5 + c])
                    plsc.store_scatter(m[s], [moff + c], vd)
                    vs = plsc.load_gather(tab_v, [s5 + c])
                    plsc.store_scatter(m[s], [moff + 5 + c], vs)
                if with_eaw:
                    eoff = (q // P1) * 128 + (q % P1) * 10
                    for c in range(5):
                        ve = plsc.load_gather(ew[s], [eoff + c])
                        plsc.store_scatter(m[s], [moff + 10 + c], ve)
                return 0

            lax.fori_loop(0, ch // 16, grp, 0)
            pltpu.async_copy(*out_copy(kk, s))

        for kk in range(max(nch - 2, 0), nch):
            pltpu.make_async_copy(*out_copy(kk, kk % 2)).wait()

    n_rows = EP // p
    return functools.partial(
        pl.kernel, mesh=mesh,
        compiler_params=pltpu.CompilerParams(needs_layout_passes=False),
        out_type=jax.ShapeDtypeStruct((n_rows * 128,), jnp.float32),
        scratch_types=scratch,
    )(body)


def _make_scatter(p, stride):
    rpc = CHS // p
    elen = rpc * 128
    nch = EWK // CHS
    mesh = plsc.VectorSubcoreMesh(core_axis_name="c", subcore_axis_name="s",
                                  num_cores=NC, num_subcores=NS)

    @functools.partial(
        pl.kernel, mesh=mesh,
        compiler_params=pltpu.CompilerParams(needs_layout_passes=False,
                                             use_tc_tiling_on_sc=False),
        out_type=jax.ShapeDtypeStruct((NC, NACC, 8), jnp.float32),
        scratch_types=[
            pltpu.VMEM((elen,), jnp.float32), pltpu.VMEM((elen,), jnp.float32),
            pltpu.VMEM((NBAT, SB), jnp.int32), pltpu.VMEM((NBAT, SB), jnp.int32),
            pltpu.VMEM((CHS, 8), jnp.float32),
            pltpu.VMEM((NROW_T, 8), jnp.float32),
            pltpu.VMEM_SHARED((NACC, 8), jnp.float32),
            pltpu.SemaphoreType.DMA, pltpu.SemaphoreType.DMA,
            pltpu.SemaphoreType.DMA,
        ],
    )
    def body(e_hbm, d_hbm, out_hbm, e0, e1, i0, i1, s_v, zo_v, acc_sh,
             sin0, sin1, sadd):
        cid = lax.axis_index("c")
        sid = lax.axis_index("s")
        wid = sid * NC + cid
        lanes = lax.iota(jnp.int32, 16)
        zero16 = jnp.zeros((16,), jnp.float32)
        e, iv, sin = [e0, e1], [i0, i1], [sin0, sin1]

        def in_copies(kk, s):
            base = wid * EWK + kk * CHS
            yield e_hbm.at[pl.ds((base // p) * 128, elen)], e[s], sin[s]
            yield d_hbm.at[pl.ds(wid * (nch * NBAT) + kk * NBAT, NBAT)], iv[s], sin[s]

        for t in in_copies(0, 0):
            pltpu.async_copy(*t)

        def zf(i, _):
            v = i * 16 + lanes
            plsc.store_scatter(zo_v, [v // 8, v % 8], zero16)
            return 0

        lax.fori_loop(0, NROW_T * 8 // 16, zf, 0)
        pltpu.sync_copy(zo_v, acc_sh.at[pl.ds(sid * NROW_T, NROW_T)])
        plsc.subcore_barrier()

        for kk in range(nch):
            s = kk % 2
            if kk + 1 < nch:
                for t in in_copies(kk + 1, 1 - s):
                    pltpu.async_copy(*t)
            for t in in_copies(kk, s):
                pltpu.make_async_copy(*t).wait()

            def batch(b, _):
                def fill(g, _):
                    t0 = b * SB + g * 16
                    q = t0 + lanes
                    eoff = (q // p) * 128 + (q % p) * stride
                    for c in range(8):
                        v = plsc.load_gather(e[s], [eoff + c])
                        plsc.store_scatter(
                            s_v, [q, jnp.full((16,), c, jnp.int32)], v)
                    return 0

                lax.fori_loop(0, SB // 16, fill, 0)
                pltpu.async_copy(s_v.at[pl.ds(b * SB, SB)],
                                 acc_sh.at[iv[s].at[b]], sadd, add=True)
                return 0

            lax.fori_loop(0, NBAT, batch, 0)

            def drain(b, _):
                pltpu.make_async_copy(s_v.at[pl.ds(b * SB, SB)],
                                      acc_sh.at[iv[s].at[b]], sadd).wait()
                return 0

            lax.fori_loop(0, NBAT, drain, 0)

        plsc.subcore_barrier()
        pltpu.sync_copy(acc_sh.at[pl.ds(sid * NROW_T, NROW_T)], zo_v)
        pltpu.sync_copy(zo_v, out_hbm.at[cid, pl.ds(sid * NROW_T, NROW_T)])

    return body


@functools.cache
def _sc_kernels():
    return (_make_gather(P1, 10, False, CH1), _make_gather(P2, 15, True, CH2),
            _make_scatter(P1, 10), _make_scatter(P2, 16))


def kernel(x, edge_index, edge_attr, params):
    src = edge_index[0]
    dst = edge_index[1]
    zpad_i = jnp.zeros((PAD_E,), jnp.int32)
    src_p = jnp.concatenate([src, zpad_i])
    dst_g = jnp.concatenate([dst, zpad_i])
    dst_s = jnp.concatenate(
        [dst, 10000 + (jnp.arange(PAD_E, dtype=jnp.int32) % 240)]).reshape(-1, SB)
    ear = jnp.concatenate([edge_attr,
                           jnp.zeros((PAD_E, 4), jnp.float32)]).reshape(R1R, 48)

    pn = params['node_encoder']
    pe = params['edge_encoder']
    ec = params['ec_layers'][0]
    hc = params['hc_layers'][0]
    pw, pb, px = params['W'], params['B'], params['X']

    def b2(v):
        return v[None, :]

    g1, g2, s1, s2 = _sc_kernels()

    r1 = ec['R1']
    r2 = hc['R1']
    o1 = ec['O']
    o2 = hc['O']
    prep_in = [
        pe[0][0], b2(pe[0][1]), pe[1][0], b2(pe[1][1]),
        r1[0][0], b2(r1[0][1]), r1[1][0], b2(r1[1][1]),
        r1[2][0], b2(r1[2][1]), r1[3][0], b2(r1[3][1]),
        pw[0][0], b2(pw[0][1]), pw[1][0], b2(pw[1][1]),
        pw[2][0], b2(pw[2][1]), pw[3][0], b2(pw[3][1]),
        r2[0][0], b2(r2[0][1]), r2[1][0], b2(r2[1][1]),
        r2[2][0], b2(r2[2][1]), r2[3][0], b2(r2[3][1]),
        o1[0][0], o2[0][0],
    ]
    pw_out = _prep_weights(prep_in)
    (we1b, be1b, we2b, be2b, w1m, w1e, b1b, w2b, b2b, w3b, b3b, w4b, b4b,
     v1b, c1b, v2b, c2b, v3b, c3b, v4b, c4b,
     w1m2, bh1, wh2, bh2, wh3, bh3, wh4, bh4,
     woh1, woa1, woh2, woa2) = pw_out

    h0 = _tc_rows(_mlp2_body, 1, [(N_NODES, 5)], NODE_BLK, NBLK_N,
                  [x], [pn[0][0], b2(pn[0][1]), pn[1][0], b2(pn[1][1])])

    m1 = g1(h0.reshape(-1), src_p, dst_g).reshape(R1R, 128)

    epad1, eaw, w12 = _tc_rows(
        _ec_edge_body, 3, [(R1R, 128), (R1R, 128), (R1R, 12)], EBLK1, NBLK1,
        [m1, ear],
        [we1b, be1b, we2b, be2b, w1m, w1e, b1b, w2b, b2b, w3b, b3b, w4b, b4b,
         v1b, c1b, v2b, c2b, v3b, c3b, v4b, c4b,
         S_E1, S_W1, S_EA1])

    agg1 = s1(epad1.reshape(-1), dst_s)[:, :N_NODES]

    h1 = _tc_rows(_node_o_body, 1, [(N_NODES, 5)], NODE_BLK, NBLK_N,
                  [h0, agg1[0], agg1[1]],
                  [woh1, woa1, b2(o1[0][1]),
                   o1[1][0], b2(o1[1][1]), o1[2][0], b2(o1[2][1]),
                   o1[3][0], b2(o1[3][1])])

    m2 = g2(h1.reshape(-1), src_p, dst_g, eaw.reshape(-1)).reshape(R2R, 128)

    epad2 = _tc_rows(
        _hc_edge_body, 1, [(R2R, 128)], EBLK2, NBLK2,
        [m2],
        [w1m2, bh1, wh2, bh2, wh3, bh3, wh4, bh4, S_E2])

    agg2 = s2(epad2.reshape(-1), dst_s)[:, :N_NODES]
    h_out, beta = _tc_rows(
        _node_final_body, 2, [(N_NODES, 2), (N_NODES, 1)], NODE_BLK, NBLK_N,
        [h1, agg2[0], agg2[1]],
        [woh2, woa2, b2(o2[0][1]),
         o2[1][0], b2(o2[1][1]), o2[2][0], b2(o2[2][1]),
         o2[3][0], b2(o2[3][1]),
         pb[0][0], b2(pb[0][1]), pb[1][0], b2(pb[1][1]),
         pb[2][0], b2(pb[2][1]), pb[3][0], b2(pb[3][1]),
         px[0][0], b2(px[0][1]), px[1][0], b2(px[1][1]),
         px[2][0], b2(px[2][1]), px[3][0], b2(px[3][1])])

    ew = w12.reshape(-1)[:N_EDGES].reshape(N_EDGES, 1)
    return (ew, h_out, beta)


# speedup vs baseline: 8.4952x; 1.1643x over previous
import functools

import numpy as np
import jax
import jax.numpy as jnp
from jax import lax
from jax.experimental import pallas as pl
from jax.experimental.pallas import tpu as pltpu
from jax.experimental.pallas import tpu_sc as plsc

N_NODES = 10000
N_EDGES = 320000
EP = 322560
PAD_E = EP - N_EDGES
P1 = 12
P2 = 8
R1R = EP // P1
R2R = EP // P2
NC, NS = 2, 16
NW = NC * NS
EWK = EP // NW
CH1 = 2016
CH2 = 1008
CHS = 2016
SB = 112
NBAT = CHS // SB
NACC = 10240
NROW_T = NACC // NS

EBLK1 = 2240
NBLK1 = R1R // EBLK1
CB1 = EBLK1 * P1
EBLK2 = 2016
NBLK2 = R2R // EBLK2
NBLK_N = 5
NODE_BLK = N_NODES // NBLK_N


def _np_sel(n_in, n_out, pairs):
    s = np.zeros((n_in, n_out), np.float32)
    for i, j in pairs:
        s[i, j] = 1.0
    return s

S_E1 = _np_sel(48, 128, [(4 * p + c, 10 * p + c) for p in range(P1) for c in range(4)])
S_W1 = _np_sel(12, 128, [(p, 10 * p) for p in range(P1)])
S_EA1 = _np_sel(48, 128, [(4 * p + c, 10 * p + 1 + c) for p in range(P1) for c in range(4)])
S_E2 = _np_sel(40, 128, [(5 * p + c, 16 * p + c) for p in range(P2) for c in range(5)])


def _relu(v):
    return jnp.maximum(v, 0.0)


def _bd(w, p):
    return jnp.kron(jnp.eye(p, dtype=jnp.float32), w)


def _bdb(b, p):
    return jnp.tile(b, p)[None, :]


def _rs(blk, d):
    return pl.BlockSpec((blk, d), lambda i: (i, 0))


def _fs(a):
    return pl.BlockSpec(a.shape, lambda i: (0,) * a.ndim)


def _mlp2_body(x, w1, b1, w2, b2, o):
    t = _relu(x[...] @ w1[...] + b1[...])
    o[...] = _relu(t @ w2[...] + b2[...])


def _ec_edge_body(m, eat, we1t, be1t, we2t, be2t, w1m, w1e, b1, w2, b2, w3, b3,
                  w4, b4, v1, c1, v2, c2, v3, c3, v4, c4,
                  se1, sw1, sea1, epad_o, eaw_o, w12_o):
    ht = _relu(we1t[...] @ eat[...] + be1t[...])
    e0t = _relu(we2t[...] @ ht + be2t[...])
    ea0 = jnp.concatenate(
        [jnp.transpose(e0t[:, p * EBLK1:(p + 1) * EBLK1]) for p in range(P1)],
        axis=1)
    t = _relu(m[...] @ w1m[...] + ea0 @ w1e[...] + b1[...])
    t = _relu(t @ w2[...] + b2[...])
    t = _relu(t @ w3[...] + b3[...])
    e48 = t @ w4[...] + b4[...]
    ea1 = ea0 + e48
    epad_o[...] = e48 @ se1[...]
    u = _relu(ea1 @ v1[...] + c1[...])
    u = _relu(u @ v2[...] + c2[...])
    u = _relu(u @ v3[...] + c3[...])
    w12 = jax.nn.sigmoid(u @ v4[...] + c4[...])
    w12_o[...] = w12
    eaw_o[...] = w12 @ sw1[...] + ea1 @ sea1[...]


def _hc_edge_body(m, w1m, b1, w2, b2, w3, b3, w4, b4, se2, epad_o):
    t = _relu(m[...] @ w1m[...] + b1[...])
    t = _relu(t @ w2[...] + b2[...])
    t = _relu(t @ w3[...] + b3[...])
    e40 = t @ w4[...] + b4[...]
    epad_o[...] = e40 @ se2[...]


def _node_o_body(h, a0, a1, wh, wa, b1, w2, b2, w3, b3, w4, b4, o):
    agg = a0[...] + a1[...]
    t = _relu(h[...] @ wh[...] + agg @ wa[...] + b1[...])
    t = _relu(t @ w2[...] + b2[...])
    t = _relu(t @ w3[...] + b3[...])
    o[...] = h[...] + (t @ w4[...] + b4[...])


def _node_final_body(h, a0, a1, wh, wa, b1, w2, b2, w3, b3, w4, b4,
                     bw1, bb1, bw2, bb2, bw3, bb3, bw4, bb4,
                     xw1, xb1, xw2, xb2, xw3, xb3, xw4, xb4,
                     hout_o, beta_o):
    agg = a0[...] + a1[...]
    t = _relu(h[...] @ wh[...] + agg @ wa[...] + b1[...])
    t = _relu(t @ w2[...] + b2[...])
    t = _relu(t @ w3[...] + b3[...])
    h2 = h[...] + (t @ w4[...] + b4[...])
    u = _relu(h2 @ bw1[...] + bb1[...])
    u = _relu(u @ bw2[...] + bb2[...])
    u = _relu(u @ bw3[...] + bb3[...])
    beta_o[...] = jax.nn.sigmoid(u @ bw4[...] + bb4[...])
    v = _relu(h2 @ xw1[...] + xb1[...])
    v = _relu(v @ xw2[...] + xb2[...])
    v = _relu(v @ xw3[...] + xb3[...])
    hout_o[...] = v @ xw4[...] + xb4[...]


def _tc_rows(body, n_out, out_shapes, blk, nblk, row_args, full_args,
             col_args=(), col_blk=0):
    in_specs = ([_rs(blk, a.shape[1]) for a in row_args]
                + [pl.BlockSpec((a.shape[0], col_blk), lambda i: (0, i))
                   for a in col_args]
                + [_fs(a) for a in full_args])
    row_args = list(row_args) + list(col_args)
    out_specs = tuple(_rs(blk, s[1]) for s in out_shapes)
    out_shape = tuple(jax.ShapeDtypeStruct(s, jnp.float32) for s in out_shapes)
    if n_out == 1:
        out_specs, out_shape = out_specs[0], out_shape[0]
    return pl.pallas_call(
        body, grid=(nblk,), in_specs=in_specs, out_specs=out_specs,
        out_shape=out_shape,
    )(*row_args, *full_args)


_N_PREP_IN = 30


def _prep_body(*refs):
    ins = refs[:_N_PREP_IN]
    outs = refs[_N_PREP_IN:]

    def bd(w, p, pad_rows=0):
        a, b = w.shape
        t = jnp.concatenate([w] * p, axis=0)
        t = jnp.concatenate([t] * p, axis=1)
        ri = lax.broadcasted_iota(jnp.int32, t.shape, 0) // a
        ci = lax.broadcasted_iota(jnp.int32, t.shape, 1) // b
        t = jnp.where(ri == ci, t, 0.0)
        if pad_rows:
            t = jnp.concatenate(
                [t, jnp.zeros((pad_rows, t.shape[1]), jnp.float32)], axis=0)
        return t

    def tl(b, p):
        return jnp.concatenate([b] * p, axis=1)

    (we1, be1, we2, be2, a1, ab1, a2, ab2, a3, ab3, a4, ab4,
     v1, vb1, v2, vb2, v3, vb3, v4, vb4,
     g1, gb1, g2, gb2, g3, gb3, g4, gb4, o1w, o2w) = [r[...] for r in ins]

    vals = [
        jnp.concatenate([jnp.transpose(we1), jnp.zeros((40, 4), jnp.float32)],
                        axis=1),
        jnp.transpose(be1), jnp.transpose(we2), jnp.transpose(be2),
        bd(a1[0:10], P1, 8), bd(a1[10:14], P1), tl(ab1, P1),
        bd(a2, P1), tl(ab2, P1), bd(a3, P1), tl(ab3, P1),
        bd(a4, P1), tl(ab4, P1),
        bd(v1, P1), tl(vb1, P1), bd(v2, P1), tl(vb2, P1),
        bd(v3, P1), tl(vb3, P1), bd(v4, P1), tl(vb4, P1),
        bd(g1, P2, 8), tl(gb1, P2), bd(g2, P2), tl(gb2, P2),
        bd(g3, P2), tl(gb3, P2), bd(g4, P2), tl(gb4, P2),
        o1w[0:5],
        jnp.concatenate([o1w[5:9], jnp.zeros((4, 40), jnp.float32)], axis=0),
        o2w[0:5],
        jnp.concatenate([o2w[5:10], jnp.zeros((3, 40), jnp.float32)], axis=0),
    ]
    for o, v in zip(outs, vals, strict=True):
        o[...] = v


def _prep_weights(ins):
    out_shapes = [
        (40, 8), (40, 1), (4, 40), (4, 1),
        (128, 480), (48, 480), (1, 480),
        (480, 480), (1, 480), (480, 480), (1, 480),
        (480, 48), (1, 48),
        (48, 480), (1, 480), (480, 480), (1, 480),
        (480, 480), (1, 480), (480, 12), (1, 12),
        (128, 320), (1, 320), (320, 320), (1, 320),
        (320, 320), (1, 320), (320, 40), (1, 40),
        (5, 40), (8, 40), (5, 40), (8, 40),
    ]
    return pl.pallas_call(
        _prep_body, grid=(1,),
        in_specs=[_fs(a) for a in ins],
        out_specs=tuple(pl.BlockSpec(s, lambda i: (0, 0)) for s in out_shapes),
        out_shape=tuple(jax.ShapeDtypeStruct(s, jnp.float32) for s in out_shapes),
    )(*ins)


def _make_gather(p, stride, with_eaw, ch):
    rpc = ch // p
    mlen = rpc * 128
    nch = EWK // ch
    ewlen = (ch // P1) * 128
    mesh = plsc.VectorSubcoreMesh(core_axis_name="c", subcore_axis_name="s",
                                  num_cores=NC, num_subcores=NS)
    scratch = [
        pltpu.VMEM((N_NODES * 5,), jnp.float32),
        pltpu.VMEM((ch,), jnp.int32), pltpu.VMEM((ch,), jnp.int32),
        pltpu.VMEM((ch,), jnp.int32), pltpu.VMEM((ch,), jnp.int32),
        pltpu.VMEM((mlen,), jnp.float32), pltpu.VMEM((mlen,), jnp.float32),
        pltpu.SemaphoreType.DMA, pltpu.SemaphoreType.DMA,
        pltpu.SemaphoreType.DMA, pltpu.SemaphoreType.DMA,
    ]
    if with_eaw:
        scratch += [pltpu.VMEM((ewlen,), jnp.float32),
                    pltpu.VMEM((ewlen,), jnp.float32)]

    def body(h_hbm, src_hbm, dst_hbm, *rest):
        if with_eaw:
            (eaw_hbm, out_hbm, tab_v, si0, si1, di0, di1, m0, m1,
             sin0, sin1, so0, so1, ew0, ew1) = rest
            ew = [ew0, ew1]
        else:
            (out_hbm, tab_v, si0, si1, di0, di1, m0, m1,
             sin0, sin1, so0, so1) = rest
            ew = None
        si, di, m = [si0, si1], [di0, di1], [m0, m1]
        sin, so = [sin0, sin1], [so0, so1]
        wid = lax.axis_index("s") * NC + lax.axis_index("c")
        lanes = lax.iota(jnp.int32, 16)
        zero16 = jnp.zeros((16,), jnp.float32)

        def in_copies(kk, s):
            base = wid * EWK + kk * ch
            yield src_hbm.at[pl.ds(base, ch)], si[s], sin[s]
            yield dst_hbm.at[pl.ds(base, ch)], di[s], sin[s]
            if with_eaw:
                yield eaw_hbm.at[pl.ds((base // P1) * 128, ewlen)], ew[s], sin[s]

        def out_copy(kk, s):
            base = wid * EWK + kk * ch
            return m[s], out_hbm.at[pl.ds((base // p) * 128, mlen)], so[s]

        for t in in_copies(0, 0):
            pltpu.async_copy(*t)
        pltpu.sync_copy(h_hbm, tab_v)

        zpad = 128 - p * stride

        def zfill(mv):
            def zf(i, _):
                v = i * 16 + lanes
                plsc.store_scatter(mv, [(v // zpad) * 128 + p * stride + (v % zpad)],
                                   zero16)
                return 0
            lax.fori_loop(0, rpc * zpad // 16, zf, 0)

        zfill(m0)
        zfill(m1)

        for kk in range(nch):
            s = kk % 2
            if kk + 1 < nch:
                for t in in_copies(kk + 1, 1 - s):
                    pltpu.async_copy(*t)
            for t in in_copies(kk, s):
                pltpu.make_async_copy(*t).wait()
            if kk >= 2:
                pltpu.make_async_copy(*out_copy(kk - 2, s)).wait()

            def grp(i, _):
                q = i * 16 + lanes
                d5 = di[s][pl.ds(i * 16, 16)] * 5
                s5 = si[s][pl.ds(i * 16, 16)] * 5
                moff = (q // p) * 128 + (q % p) * stride
                for c in range(5):
                    vd = plsc.load_gather(tab_v, [d5 + c])
                    plsc.store_scatter(m[s], [moff + c], vd)
                    vs = plsc.load_gather(tab_v, [s5 + c])
                    plsc.store_scatter(m[s], [moff + 5 + c], vs)
                if with_eaw:
                    eoff = (q // P1) * 128 + (q % P1) * 10
                    for c in range(5):
                        ve = plsc.load_gather(ew[s], [eoff + c])
                        plsc.store_scatter(m[s], [moff + 10 + c], ve)
                return 0

            lax.fori_loop(0, ch // 16, grp, 0)
            pltpu.async_copy(*out_copy(kk, s))

        for kk in range(max(nch - 2, 0), nch):
            pltpu.make_async_copy(*out_copy(kk, kk % 2)).wait()

    n_rows = EP // p
    return functools.partial(
        pl.kernel, mesh=mesh,
        compiler_params=pltpu.CompilerParams(needs_layout_passes=False),
        out_type=jax.ShapeDtypeStruct((n_rows * 128,), jnp.float32),
        scratch_types=scratch,
    )(body)


def _make_scatter(p, stride):
    rpc = CHS // p
    elen = rpc * 128
    nch = EWK // CHS
    mesh = plsc.VectorSubcoreMesh(core_axis_name="c", subcore_axis_name="s",
                                  num_cores=NC, num_subcores=NS)

    @functools.partial(
        pl.kernel, mesh=mesh,
        compiler_params=pltpu.CompilerParams(needs_layout_passes=False,
                                             use_tc_tiling_on_sc=False),
        out_type=jax.ShapeDtypeStruct((NC, NACC, 8), jnp.float32),
        scratch_types=[
            pltpu.VMEM((elen,), jnp.float32), pltpu.VMEM((elen,), jnp.float32),
            pltpu.VMEM((NBAT, SB), jnp.int32), pltpu.VMEM((NBAT, SB), jnp.int32),
            pltpu.VMEM((CHS, 8), jnp.float32),
            pltpu.VMEM((NROW_T, 8), jnp.float32),
            pltpu.VMEM_SHARED((NACC, 8), jnp.float32),
            pltpu.SemaphoreType.DMA, pltpu.SemaphoreType.DMA,
            pltpu.SemaphoreType.DMA,
        ],
    )
    def body(e_hbm, d_hbm, out_hbm, e0, e1, i0, i1, s_v, zo_v, acc_sh,
             sin0, sin1, sadd):
        cid = lax.axis_index("c")
        sid = lax.axis_index("s")
        wid = sid * NC + cid
        lanes = lax.iota(jnp.int32, 16)
        zero16 = jnp.zeros((16,), jnp.float32)
        e, iv, sin = [e0, e1], [i0, i1], [sin0, sin1]

        def in_copies(kk, s):
            base = wid * EWK + kk * CHS
            yield e_hbm.at[pl.ds((base // p) * 128, elen)], e[s], sin[s]
            yield d_hbm.at[pl.ds(wid * (nch * NBAT) + kk * NBAT, NBAT)], iv[s], sin[s]

        for t in in_copies(0, 0):
            pltpu.async_copy(*t)

        def zf(i, _):
            v = i * 16 + lanes
            plsc.store_scatter(zo_v, [v // 8, v % 8], zero16)
            return 0

        lax.fori_loop(0, NROW_T * 8 // 16, zf, 0)
        pltpu.sync_copy(zo_v, acc_sh.at[pl.ds(sid * NROW_T, NROW_T)])
        plsc.subcore_barrier()

        for kk in range(nch):
            s = kk % 2
            if kk + 1 < nch:
                for t in in_copies(kk + 1, 1 - s):
                    pltpu.async_copy(*t)
            for t in in_copies(kk, s):
                pltpu.make_async_copy(*t).wait()

            def batch(b, _):
                def fill(g, _):
                    t0 = b * SB + g * 16
                    q = t0 + lanes
                    eoff = (q // p) * 128 + (q % p) * stride
                    for c in range(8):
                        v = plsc.load_gather(e[s], [eoff + c])
                        plsc.store_scatter(
                            s_v, [q, jnp.full((16,), c, jnp.int32)], v)
                    return 0

                lax.fori_loop(0, SB // 16, fill, 0)
                pltpu.async_copy(s_v.at[pl.ds(b * SB, SB)],
                                 acc_sh.at[iv[s].at[b]], sadd, add=True)
                return 0

            lax.fori_loop(0, NBAT, batch, 0)

            def drain(b, _):
                pltpu.make_async_copy(s_v.at[pl.ds(b * SB, SB)],
                                      acc_sh.at[iv[s].at[b]], sadd).wait()
                return 0

            lax.fori_loop(0, NBAT, drain, 0)

        plsc.subcore_barrier()
        pltpu.sync_copy(acc_sh.at[pl.ds(sid * NROW_T, NROW_T)], zo_v)
        pltpu.sync_copy(zo_v, out_hbm.at[cid, pl.ds(sid * NROW_T, NROW_T)])

    return body


@functools.cache
def _sc_kernels():
    return (_make_gather(P1, 10, False, CH1), _make_gather(P2, 15, True, CH2),
            _make_scatter(P1, 10), _make_scatter(P2, 16))


def kernel(x, edge_index, edge_attr, params):
    src = edge_index[0]
    dst = edge_index[1]

    def permute(a):
        return a.reshape(NBLK1, P1, EBLK1).transpose(0, 2, 1).reshape(-1)

    zpad_i = jnp.zeros((PAD_E,), jnp.int32)
    src_p = permute(jnp.concatenate([src, zpad_i]))
    dst_g = permute(jnp.concatenate([dst, zpad_i]))
    dst_s = permute(jnp.concatenate(
        [dst, 10000 + (jnp.arange(PAD_E, dtype=jnp.int32) % 240)])).reshape(-1, SB)
    eat8 = jnp.pad(edge_attr.T, ((0, 4), (0, PAD_E)))

    pn = params['node_encoder']
    pe = params['edge_encoder']
    ec = params['ec_layers'][0]
    hc = params['hc_layers'][0]
    pw, pb, px = params['W'], params['B'], params['X']

    def b2(v):
        return v[None, :]

    g1, g2, s1, s2 = _sc_kernels()

    r1 = ec['R1']
    r2 = hc['R1']
    o1 = ec['O']
    o2 = hc['O']
    prep_in = [
        pe[0][0], b2(pe[0][1]), pe[1][0], b2(pe[1][1]),
        r1[0][0], b2(r1[0][1]), r1[1][0], b2(r1[1][1]),
        r1[2][0], b2(r1[2][1]), r1[3][0], b2(r1[3][1]),
        pw[0][0], b2(pw[0][1]), pw[1][0], b2(pw[1][1]),
        pw[2][0], b2(pw[2][1]), pw[3][0], b2(pw[3][1]),
        r2[0][0], b2(r2[0][1]), r2[1][0], b2(r2[1][1]),
        r2[2][0], b2(r2[2][1]), r2[3][0], b2(r2[3][1]),
        o1[0][0], o2[0][0],
    ]
    pw_out = _prep_weights(prep_in)
    (we1b, be1b, we2b, be2b, w1m, w1e, b1b, w2b, b2b, w3b, b3b, w4b, b4b,
     v1b, c1b, v2b, c2b, v3b, c3b, v4b, c4b,
     w1m2, bh1, wh2, bh2, wh3, bh3, wh4, bh4,
     woh1, woa1, woh2, woa2) = pw_out

    h0 = _tc_rows(_mlp2_body, 1, [(N_NODES, 5)], NODE_BLK, NBLK_N,
                  [x], [pn[0][0], b2(pn[0][1]), pn[1][0], b2(pn[1][1])])

    m1 = g1(h0.reshape(-1), src_p, dst_g).reshape(R1R, 128)

    epad1, eaw, w12 = _tc_rows(
        _ec_edge_body, 3, [(R1R, 128), (R1R, 128), (R1R, 12)], EBLK1, NBLK1,
        [m1],
        [we1b, be1b, we2b, be2b, w1m, w1e, b1b, w2b, b2b, w3b, b3b, w4b, b4b,
         v1b, c1b, v2b, c2b, v3b, c3b, v4b, c4b,
         S_E1, S_W1, S_EA1],
        col_args=[eat8], col_blk=CB1)

    agg1 = s1(epad1.reshape(-1), dst_s)[:, :N_NODES]

    h1 = _tc_rows(_node_o_body, 1, [(N_NODES, 5)], NODE_BLK, NBLK_N,
                  [h0, agg1[0], agg1[1]],
                  [woh1, woa1, b2(o1[0][1]),
                   o1[1][0], b2(o1[1][1]), o1[2][0], b2(o1[2][1]),
                   o1[3][0], b2(o1[3][1])])

    m2 = g2(h1.reshape(-1), src_p, dst_g, eaw.reshape(-1)).reshape(R2R, 128)

    epad2 = _tc_rows(
        _hc_edge_body, 1, [(R2R, 128)], EBLK2, NBLK2,
        [m2],
        [w1m2, bh1, wh2, bh2, wh3, bh3, wh4, bh4, S_E2])

    agg2 = s2(epad2.reshape(-1), dst_s)[:, :N_NODES]
    h_out, beta = _tc_rows(
        _node_final_body, 2, [(N_NODES, 2), (N_NODES, 1)], NODE_BLK, NBLK_N,
        [h1, agg2[0], agg2[1]],
        [woh2, woa2, b2(o2[0][1]),
         o2[1][0], b2(o2[1][1]), o2[2][0], b2(o2[2][1]),
         o2[3][0], b2(o2[3][1]),
         pb[0][0], b2(pb[0][1]), pb[1][0], b2(pb[1][1]),
         pb[2][0], b2(pb[2][1]), pb[3][0], b2(pb[3][1]),
         px[0][0], b2(px[0][1]), px[1][0], b2(px[1][1]),
         px[2][0], b2(px[2][1]), px[3][0], b2(px[3][1])])

    ew = (w12.reshape(NBLK1, EBLK1, P1).transpose(0, 2, 1)
          .reshape(-1)[:N_EDGES].reshape(N_EDGES, 1))
    return (ew, h_out, beta)


# speedup vs baseline: 9.1056x; 1.0719x over previous
import functools

import numpy as np
import jax
import jax.numpy as jnp
from jax import lax
from jax.experimental import pallas as pl
from jax.experimental.pallas import tpu as pltpu
from jax.experimental.pallas import tpu_sc as plsc

N_NODES = 10000
N_EDGES = 320000
EP = 322560
PAD_E = EP - N_EDGES
P1 = 12
P2 = 8
R1R = EP // P1
R2R = EP // P2
NC, NS = 2, 16
NW = NC * NS
EWK = EP // NW
CH1 = 2016
CH2 = 1008
CHS = 2016
SB = 112
NBAT = CHS // SB
NACC = 10240
NROW_T = NACC // NS

EBLK1 = 2240
NBLK1 = R1R // EBLK1
CB1 = EBLK1 * P1
EBLK2 = 2016
NBLK2 = R2R // EBLK2
NBLK_N = 5
NODE_BLK = N_NODES // NBLK_N


def _np_sel(n_in, n_out, pairs):
    s = np.zeros((n_in, n_out), np.float32)
    for i, j in pairs:
        s[i, j] = 1.0
    return s

S_E1 = _np_sel(48, 128, [(4 * p + c, 10 * p + c) for p in range(P1) for c in range(4)])
S_W1 = _np_sel(12, 128, [(p, 10 * p) for p in range(P1)])
S_EA1 = _np_sel(48, 128, [(4 * p + c, 10 * p + 1 + c) for p in range(P1) for c in range(4)])
S_E2 = _np_sel(40, 128, [(5 * p + c, 16 * p + c) for p in range(P2) for c in range(5)])


def _relu(v):
    return jnp.maximum(v, 0.0)


def _bd(w, p):
    return jnp.kron(jnp.eye(p, dtype=jnp.float32), w)


def _bdb(b, p):
    return jnp.tile(b, p)[None, :]


def _rs(blk, d):
    return pl.BlockSpec((blk, d), lambda i: (i, 0))


def _fs(a):
    return pl.BlockSpec(a.shape, lambda i: (0,) * a.ndim)


def _mlp2_body(x, w1, b1, w2, b2, o):
    t = _relu(x[...] @ w1[...] + b1[...])
    o[...] = _relu(t @ w2[...] + b2[...])


def _ec_edge_body(m, eat, we1t, be1t, we2t, be2t, w1m, w1e, b1, w2, b2, w3, b3,
                  w4, b4, v1, c1, v2, c2, v3, c3, v4, c4,
                  se1, sw1, sea1, epad_o, eaw_o, w12_o):
    ht = _relu(we1t[...] @ eat[...] + be1t[...])
    e0t = _relu(we2t[...] @ ht + be2t[...])
    ea0 = jnp.concatenate(
        [jnp.transpose(e0t[:, p * EBLK1:(p + 1) * EBLK1]) for p in range(P1)],
        axis=1)
    t = _relu(m[...] @ w1m[...] + ea0 @ w1e[...] + b1[...])
    t = _relu(t @ w2[...] + b2[...])
    t = _relu(t @ w3[...] + b3[...])
    e48 = t @ w4[...] + b4[...]
    ea1 = ea0 + e48
    epad_o[...] = e48 @ se1[...]
    u = _relu(ea1 @ v1[...] + c1[...])
    u = _relu(u @ v2[...] + c2[...])
    u = _relu(u @ v3[...] + c3[...])
    w12 = jax.nn.sigmoid(u @ v4[...] + c4[...])
    w12_o[...] = w12
    eaw_o[...] = w12 @ sw1[...] + ea1 @ sea1[...]


def _hc_edge_body(m, w1m, b1, w2, b2, w3, b3, w4, b4, se2, epad_o):
    t = _relu(m[...] @ w1m[...] + b1[...])
    t = _relu(t @ w2[...] + b2[...])
    t = _relu(t @ w3[...] + b3[...])
    e40 = t @ w4[...] + b4[...]
    epad_o[...] = e40 @ se2[...]


def _node_o_body(h, a0, a1, wh, wa, b1, w2, b2, w3, b3, w4, b4, o):
    agg = a0[...] + a1[...]
    t = _relu(h[...] @ wh[...] + agg @ wa[...] + b1[...])
    t = _relu(t @ w2[...] + b2[...])
    t = _relu(t @ w3[...] + b3[...])
    o[...] = h[...] + (t @ w4[...] + b4[...])


def _node_final_body(h, a0, a1, wh, wa, b1, w2, b2, w3, b3, w4, b4,
                     bw1, bb1, bw2, bb2, bw3, bb3, bw4, bb4,
                     xw1, xb1, xw2, xb2, xw3, xb3, xw4, xb4,
                     hout_o, beta_o):
    agg = a0[...] + a1[...]
    t = _relu(h[...] @ wh[...] + agg @ wa[...] + b1[...])
    t = _relu(t @ w2[...] + b2[...])
    t = _relu(t @ w3[...] + b3[...])
    h2 = h[...] + (t @ w4[...] + b4[...])
    u = _relu(h2 @ bw1[...] + bb1[...])
    u = _relu(u @ bw2[...] + bb2[...])
    u = _relu(u @ bw3[...] + bb3[...])
    beta_o[...] = jax.nn.sigmoid(u @ bw4[...] + bb4[...])
    v = _relu(h2 @ xw1[...] + xb1[...])
    v = _relu(v @ xw2[...] + xb2[...])
    v = _relu(v @ xw3[...] + xb3[...])
    hout_o[...] = v @ xw4[...] + xb4[...]


def _tc_rows(body, n_out, out_shapes, blk, nblk, row_args, full_args,
             col_args=(), col_blk=0):
    in_specs = ([_rs(blk, a.shape[1]) for a in row_args]
                + [pl.BlockSpec((a.shape[0], col_blk), lambda i: (0, i))
                   for a in col_args]
                + [_fs(a) for a in full_args])
    row_args = list(row_args) + list(col_args)
    out_specs = tuple(_rs(blk, s[1]) for s in out_shapes)
    out_shape = tuple(jax.ShapeDtypeStruct(s, jnp.float32) for s in out_shapes)
    if n_out == 1:
        out_specs, out_shape = out_specs[0], out_shape[0]
    return pl.pallas_call(
        body, grid=(nblk,), in_specs=in_specs, out_specs=out_specs,
        out_shape=out_shape,
    )(*row_args, *full_args)


_N_PREP_IN = 30


def _prep_body(*refs):
    ins = refs[:_N_PREP_IN]
    outs = refs[_N_PREP_IN:]

    def bd(w, p, pad_rows=0):
        a, b = w.shape
        t = jnp.concatenate([w] * p, axis=0)
        t = jnp.concatenate([t] * p, axis=1)
        ri = lax.broadcasted_iota(jnp.int32, t.shape, 0) // a
        ci = lax.broadcasted_iota(jnp.int32, t.shape, 1) // b
        t = jnp.where(ri == ci, t, 0.0)
        if pad_rows:
            t = jnp.concatenate(
                [t, jnp.zeros((pad_rows, t.shape[1]), jnp.float32)], axis=0)
        return t

    def tl(b, p):
        return jnp.concatenate([b] * p, axis=1)

    (we1, be1, we2, be2, a1, ab1, a2, ab2, a3, ab3, a4, ab4,
     v1, vb1, v2, vb2, v3, vb3, v4, vb4,
     g1, gb1, g2, gb2, g3, gb3, g4, gb4, o1w, o2w) = [r[...] for r in ins]

    vals = [
        jnp.concatenate([jnp.transpose(we1), jnp.zeros((40, 4), jnp.float32)],
                        axis=1),
        jnp.transpose(be1), jnp.transpose(we2), jnp.transpose(be2),
        bd(a1[0:10], P1, 8), bd(a1[10:14], P1), tl(ab1, P1),
        bd(a2, P1), tl(ab2, P1), bd(a3, P1), tl(ab3, P1),
        bd(a4, P1), tl(ab4, P1),
        bd(v1, P1), tl(vb1, P1), bd(v2, P1), tl(vb2, P1),
        bd(v3, P1), tl(vb3, P1), bd(v4, P1), tl(vb4, P1),
        bd(g1, P2, 8), tl(gb1, P2), bd(g2, P2), tl(gb2, P2),
        bd(g3, P2), tl(gb3, P2), bd(g4, P2), tl(gb4, P2),
        o1w[0:5],
        jnp.concatenate([o1w[5:9], jnp.zeros((4, 40), jnp.float32)], axis=0),
        o2w[0:5],
        jnp.concatenate([o2w[5:10], jnp.zeros((3, 40), jnp.float32)], axis=0),
    ]
    for o, v in zip(outs, vals, strict=True):
        o[...] = v


def _prep_weights(ins):
    out_shapes = [
        (40, 8), (40, 1), (4, 40), (4, 1),
        (128, 480), (48, 480), (1, 480),
        (480, 480), (1, 480), (480, 480), (1, 480),
        (480, 48), (1, 48),
        (48, 480), (1, 480), (480, 480), (1, 480),
        (480, 480), (1, 480), (480, 12), (1, 12),
        (128, 320), (1, 320), (320, 320), (1, 320),
        (320, 320), (1, 320), (320, 40), (1, 40),
        (5, 40), (8, 40), (5, 40), (8, 40),
    ]
    return pl.pallas_call(
        _prep_body, grid=(1,),
        in_specs=[_fs(a) for a in ins],
        out_specs=tuple(pl.BlockSpec(s, lambda i: (0, 0)) for s in out_shapes),
        out_shape=tuple(jax.ShapeDtypeStruct(s, jnp.float32) for s in out_shapes),
    )(*ins)


def _make_gather(p, stride, with_eaw, ch):
    rpc = ch // p
    mlen = rpc * 128
    nch = EWK // ch
    ewlen = (ch // P1) * 128
    mesh = plsc.VectorSubcoreMesh(core_axis_name="c", subcore_axis_name="s",
                                  num_cores=NC, num_subcores=NS)
    scratch = [
        pltpu.VMEM((N_NODES * 5,), jnp.float32),
        pltpu.VMEM((ch,), jnp.int32), pltpu.VMEM((ch,), jnp.int32),
        pltpu.VMEM((ch,), jnp.int32), pltpu.VMEM((ch,), jnp.int32),
        pltpu.VMEM((mlen,), jnp.float32), pltpu.VMEM((mlen,), jnp.float32),
        pltpu.SemaphoreType.DMA, pltpu.SemaphoreType.DMA,
        pltpu.SemaphoreType.DMA, pltpu.SemaphoreType.DMA,
    ]
    if with_eaw:
        scratch += [pltpu.VMEM((ewlen,), jnp.float32),
                    pltpu.VMEM((ewlen,), jnp.float32)]

    def body(h_hbm, src_hbm, dst_hbm, *rest):
        if with_eaw:
            (eaw_hbm, out_hbm, tab_v, si0, si1, di0, di1, m0, m1,
             sin0, sin1, so0, so1, ew0, ew1) = rest
            ew = [ew0, ew1]
        else:
            (out_hbm, tab_v, si0, si1, di0, di1, m0, m1,
             sin0, sin1, so0, so1) = rest
            ew = None
        si, di, m = [si0, si1], [di0, di1], [m0, m1]
        sin, so = [sin0, sin1], [so0, so1]
        wid = lax.axis_index("s") * NC + lax.axis_index("c")
        lanes = lax.iota(jnp.int32, 16)
        zero16 = jnp.zeros((16,), jnp.float32)

        def in_copies(kk, s):
            base = wid * EWK + kk * ch
            yield src_hbm.at[pl.ds(base, ch)], si[s], sin[s]
            yield dst_hbm.at[pl.ds(base, ch)], di[s], sin[s]
            if with_eaw:
                yield eaw_hbm.at[pl.ds((base // P1) * 128, ewlen)], ew[s], sin[s]

        def out_copy(kk, s):
            base = wid * EWK + kk * ch
            return m[s], out_hbm.at[pl.ds((base // p) * 128, mlen)], so[s]

        for t in in_copies(0, 0):
            pltpu.async_copy(*t)
        pltpu.sync_copy(h_hbm, tab_v)

        zpad = 128 - p * stride

        def zfill(mv):
            @plsc.parallel_loop(0, rpc * zpad // 16, unroll=8)
            def _zf(i):
                v = i * 16 + lanes
                plsc.store_scatter(mv, [(v // zpad) * 128 + p * stride + (v % zpad)],
                                   zero16)

        zfill(m0)
        zfill(m1)

        for kk in range(nch):
            s = kk % 2
            if kk + 1 < nch:
                for t in in_copies(kk + 1, 1 - s):
                    pltpu.async_copy(*t)
            for t in in_copies(kk, s):
                pltpu.make_async_copy(*t).wait()
            if kk >= 2:
                pltpu.make_async_copy(*out_copy(kk - 2, s)).wait()

            @plsc.parallel_loop(0, ch // 16, unroll=4)
            def _grp(i):
                q = i * 16 + lanes
                d5 = di[s][pl.ds(i * 16, 16)] * 5
                s5 = si[s][pl.ds(i * 16, 16)] * 5
                moff = (q // p) * 128 + (q % p) * stride
                for c in range(5):
                    vd = plsc.load_gather(tab_v, [d5 + c])
                    plsc.store_scatter(m[s], [moff + c], vd)
                    vs = plsc.load_gather(tab_v, [s5 + c])
                    plsc.store_scatter(m[s], [moff + 5 + c], vs)
                if with_eaw:
                    eoff = (q // P1) * 128 + (q % P1) * 10
                    for c in range(5):
                        ve = plsc.load_gather(ew[s], [eoff + c])
                        plsc.store_scatter(m[s], [moff + 10 + c], ve)
            pltpu.async_copy(*out_copy(kk, s))

        for kk in range(max(nch - 2, 0), nch):
            pltpu.make_async_copy(*out_copy(kk, kk % 2)).wait()

    n_rows = EP // p
    return functools.partial(
        pl.kernel, mesh=mesh,
        compiler_params=pltpu.CompilerParams(needs_layout_passes=False),
        out_type=jax.ShapeDtypeStruct((n_rows * 128,), jnp.float32),
        scratch_types=scratch,
    )(body)


def _make_scatter(p, stride):
    rpc = CHS // p
    elen = rpc * 128
    nch = EWK // CHS
    mesh = plsc.VectorSubcoreMesh(core_axis_name="c", subcore_axis_name="s",
                                  num_cores=NC, num_subcores=NS)

    @functools.partial(
        pl.kernel, mesh=mesh,
        compiler_params=pltpu.CompilerParams(needs_layout_passes=False,
                                             use_tc_tiling_on_sc=False),
        out_type=jax.ShapeDtypeStruct((NC, NACC, 8), jnp.float32),
        scratch_types=[
            pltpu.VMEM((elen,), jnp.float32), pltpu.VMEM((elen,), jnp.float32),
            pltpu.VMEM((NBAT, SB), jnp.int32), pltpu.VMEM((NBAT, SB), jnp.int32),
            pltpu.VMEM((CHS, 8), jnp.float32),
            pltpu.VMEM((NROW_T, 8), jnp.float32),
            pltpu.VMEM_SHARED((NACC, 8), jnp.float32),
            pltpu.SemaphoreType.DMA, pltpu.SemaphoreType.DMA,
            pltpu.SemaphoreType.DMA,
        ],
    )
    def body(e_hbm, d_hbm, out_hbm, e0, e1, i0, i1, s_v, zo_v, acc_sh,
             sin0, sin1, sadd):
        cid = lax.axis_index("c")
        sid = lax.axis_index("s")
        wid = sid * NC + cid
        lanes = lax.iota(jnp.int32, 16)
        zero16 = jnp.zeros((16,), jnp.float32)
        e, iv, sin = [e0, e1], [i0, i1], [sin0, sin1]

        def in_copies(kk, s):
            base = wid * EWK + kk * CHS
            yield e_hbm.at[pl.ds((base // p) * 128, elen)], e[s], sin[s]
            yield d_hbm.at[pl.ds(wid * (nch * NBAT) + kk * NBAT, NBAT)], iv[s], sin[s]

        for t in in_copies(0, 0):
            pltpu.async_copy(*t)

        @plsc.parallel_loop(0, NROW_T * 8 // 16, unroll=8)
        def _zf(i):
            v = i * 16 + lanes
            plsc.store_scatter(zo_v, [v // 8, v % 8], zero16)
        pltpu.sync_copy(zo_v, acc_sh.at[pl.ds(sid * NROW_T, NROW_T)])
        plsc.subcore_barrier()

        for kk in range(nch):
            s = kk % 2
            if kk + 1 < nch:
                for t in in_copies(kk + 1, 1 - s):
                    pltpu.async_copy(*t)
            for t in in_copies(kk, s):
                pltpu.make_async_copy(*t).wait()

            def batch(b, _):
                @plsc.parallel_loop(0, SB // 16, unroll=7)
                def _fill(g):
                    t0 = b * SB + g * 16
                    q = t0 + lanes
                    eoff = (q // p) * 128 + (q % p) * stride
                    for c in range(8):
                        v = plsc.load_gather(e[s], [eoff + c])
                        plsc.store_scatter(
                            s_v, [q, jnp.full((16,), c, jnp.int32)], v)
                pltpu.async_copy(s_v.at[pl.ds(b * SB, SB)],
                                 acc_sh.at[iv[s].at[b]], sadd, add=True)
                return 0

            lax.fori_loop(0, NBAT, batch, 0)

            def drain(b, _):
                pltpu.make_async_copy(s_v.at[pl.ds(b * SB, SB)],
                                      acc_sh.at[iv[s].at[b]], sadd).wait()
                return 0

            lax.fori_loop(0, NBAT, drain, 0)

        plsc.subcore_barrier()
        pltpu.sync_copy(acc_sh.at[pl.ds(sid * NROW_T, NROW_T)], zo_v)
        pltpu.sync_copy(zo_v, out_hbm.at[cid, pl.ds(sid * NROW_T, NROW_T)])

    return body


@functools.cache
def _sc_kernels():
    return (_make_gather(P1, 10, False, CH1), _make_gather(P2, 15, True, CH2),
            _make_scatter(P1, 10), _make_scatter(P2, 16))


def kernel(x, edge_index, edge_attr, params):
    src = edge_index[0]
    dst = edge_index[1]

    def permute(a):
        return a.reshape(NBLK1, P1, EBLK1).transpose(0, 2, 1).reshape(-1)

    zpad_i = jnp.zeros((PAD_E,), jnp.int32)
    src_p = permute(jnp.concatenate([src, zpad_i]))
    dst_g = permute(jnp.concatenate([dst, zpad_i]))
    dst_s = permute(jnp.concatenate(
        [dst, 10000 + (jnp.arange(PAD_E, dtype=jnp.int32) % 240)])).reshape(-1, SB)
    eat8 = jnp.pad(edge_attr.T, ((0, 4), (0, PAD_E)))

    pn = params['node_encoder']
    pe = params['edge_encoder']
    ec = params['ec_layers'][0]
    hc = params['hc_layers'][0]
    pw, pb, px = params['W'], params['B'], params['X']

    def b2(v):
        return v[None, :]

    g1, g2, s1, s2 = _sc_kernels()

    r1 = ec['R1']
    r2 = hc['R1']
    o1 = ec['O']
    o2 = hc['O']
    prep_in = [
        pe[0][0], b2(pe[0][1]), pe[1][0], b2(pe[1][1]),
        r1[0][0], b2(r1[0][1]), r1[1][0], b2(r1[1][1]),
        r1[2][0], b2(r1[2][1]), r1[3][0], b2(r1[3][1]),
        pw[0][0], b2(pw[0][1]), pw[1][0], b2(pw[1][1]),
        pw[2][0], b2(pw[2][1]), pw[3][0], b2(pw[3][1]),
        r2[0][0], b2(r2[0][1]), r2[1][0], b2(r2[1][1]),
        r2[2][0], b2(r2[2][1]), r2[3][0], b2(r2[3][1]),
        o1[0][0], o2[0][0],
    ]
    pw_out = _prep_weights(prep_in)
    (we1b, be1b, we2b, be2b, w1m, w1e, b1b, w2b, b2b, w3b, b3b, w4b, b4b,
     v1b, c1b, v2b, c2b, v3b, c3b, v4b, c4b,
     w1m2, bh1, wh2, bh2, wh3, bh3, wh4, bh4,
     woh1, woa1, woh2, woa2) = pw_out

    h0 = _tc_rows(_mlp2_body, 1, [(N_NODES, 5)], NODE_BLK, NBLK_N,
                  [x], [pn[0][0], b2(pn[0][1]), pn[1][0], b2(pn[1][1])])

    m1 = g1(h0.reshape(-1), src_p, dst_g).reshape(R1R, 128)

    epad1, eaw, w12 = _tc_rows(
        _ec_edge_body, 3, [(R1R, 128), (R1R, 128), (R1R, 12)], EBLK1, NBLK1,
        [m1],
        [we1b, be1b, we2b, be2b, w1m, w1e, b1b, w2b, b2b, w3b, b3b, w4b, b4b,
         v1b, c1b, v2b, c2b, v3b, c3b, v4b, c4b,
         S_E1, S_W1, S_EA1],
        col_args=[eat8], col_blk=CB1)

    agg1 = s1(epad1.reshape(-1), dst_s)[:, :N_NODES]

    h1 = _tc_rows(_node_o_body, 1, [(N_NODES, 5)], NODE_BLK, NBLK_N,
                  [h0, agg1[0], agg1[1]],
                  [woh1, woa1, b2(o1[0][1]),
                   o1[1][0], b2(o1[1][1]), o1[2][0], b2(o1[2][1]),
                   o1[3][0], b2(o1[3][1])])

    m2 = g2(h1.reshape(-1), src_p, dst_g, eaw.reshape(-1)).reshape(R2R, 128)

    epad2 = _tc_rows(
        _hc_edge_body, 1, [(R2R, 128)], EBLK2, NBLK2,
        [m2],
        [w1m2, bh1, wh2, bh2, wh3, bh3, wh4, bh4, S_E2])

    agg2 = s2(epad2.reshape(-1), dst_s)[:, :N_NODES]
    h_out, beta = _tc_rows(
        _node_final_body, 2, [(N_NODES, 2), (N_NODES, 1)], NODE_BLK, NBLK_N,
        [h1, agg2[0], agg2[1]],
        [woh2, woa2, b2(o2[0][1]),
         o2[1][0], b2(o2[1][1]), o2[2][0], b2(o2[2][1]),
         o2[3][0], b2(o2[3][1]),
         pb[0][0], b2(pb[0][1]), pb[1][0], b2(pb[1][1]),
         pb[2][0], b2(pb[2][1]), pb[3][0], b2(pb[3][1]),
         px[0][0], b2(px[0][1]), px[1][0], b2(px[1][1]),
         px[2][0], b2(px[2][1]), px[3][0], b2(px[3][1])])

    ew = (w12.reshape(NBLK1, EBLK1, P1).transpose(0, 2, 1)
          .reshape(-1)[:N_EDGES].reshape(N_EDGES, 1))
    return (ew, h_out, beta)


# speedup vs baseline: 9.1282x; 1.0025x over previous
import functools

import numpy as np
import jax
import jax.numpy as jnp
from jax import lax
from jax.experimental import pallas as pl
from jax.experimental.pallas import tpu as pltpu
from jax.experimental.pallas import tpu_sc as plsc

N_NODES = 10000
N_EDGES = 320000
EP = 322560
PAD_E = EP - N_EDGES
P1 = 12
P2 = 8
R1R = EP // P1
R2R = EP // P2
NC, NS = 2, 16
NW = NC * NS
EWK = EP // NW
CH1 = 2016
CH2 = 1008
CHS = 2016
SB = 112
NBAT = CHS // SB
NACC = 10240
NROW_T = NACC // NS

EBLK1 = 2240
NBLK1 = R1R // EBLK1
CB1 = EBLK1 * P1
EBLK2 = 2016
NBLK2 = R2R // EBLK2
NBLK_N = 5
NODE_BLK = N_NODES // NBLK_N


def _np_sel(n_in, n_out, pairs):
    s = np.zeros((n_in, n_out), np.float32)
    for i, j in pairs:
        s[i, j] = 1.0
    return s

S_E1 = _np_sel(48, 128, [(4 * p + c, 10 * p + c) for p in range(P1) for c in range(4)])
S_W1 = _np_sel(12, 128, [(p, 10 * p) for p in range(P1)])
S_EA1 = _np_sel(48, 128, [(4 * p + c, 10 * p + 1 + c) for p in range(P1) for c in range(4)])
S_E2 = _np_sel(40, 128, [(5 * p + c, 16 * p + c) for p in range(P2) for c in range(5)])


def _relu(v):
    return jnp.maximum(v, 0.0)


def _mm(x, w):
    return jnp.dot(x.astype(jnp.bfloat16), w.astype(jnp.bfloat16),
                   preferred_element_type=jnp.float32)


def _bd(w, p):
    return jnp.kron(jnp.eye(p, dtype=jnp.float32), w)


def _bdb(b, p):
    return jnp.tile(b, p)[None, :]


def _rs(blk, d):
    return pl.BlockSpec((blk, d), lambda i: (i, 0))


def _fs(a):
    return pl.BlockSpec(a.shape, lambda i: (0,) * a.ndim)


def _mlp2_body(x, w1, b1, w2, b2, o):
    t = _relu(x[...] @ w1[...] + b1[...])
    o[...] = _relu(t @ w2[...] + b2[...])


def _ec_edge_body(m, eat, we1t, be1t, we2t, be2t, w1m, w1e, b1, w2, b2, w3, b3,
                  w4, b4, v1, c1, v2, c2, v3, c3, v4, c4,
                  se1, sw1, sea1, epad_o, eaw_o, w12_o):
    ht = _relu(we1t[...] @ eat[...] + be1t[...])
    e0t = _relu(we2t[...] @ ht + be2t[...])
    ea0 = jnp.concatenate(
        [jnp.transpose(e0t[:, p * EBLK1:(p + 1) * EBLK1]) for p in range(P1)],
        axis=1)
    t = _relu(_mm(m[...], w1m[...]) + ea0 @ w1e[...] + b1[...])
    t = _relu(_mm(t, w2[...]) + b2[...])
    t = _relu(_mm(t, w3[...]) + b3[...])
    e48 = _mm(t, w4[...]) + b4[...]
    ea1 = ea0 + e48
    epad_o[...] = e48 @ se1[...]
    u = _relu(ea1 @ v1[...] + c1[...])
    u = _relu(_mm(u, v2[...]) + c2[...])
    u = _relu(_mm(u, v3[...]) + c3[...])
    w12 = jax.nn.sigmoid(_mm(u, v4[...]) + c4[...])
    w12_o[...] = w12
    eaw_o[...] = w12 @ sw1[...] + ea1 @ sea1[...]


def _hc_edge_body(m, w1m, b1, w2, b2, w3, b3, w4, b4, se2, epad_o):
    t = _relu(_mm(m[...], w1m[...]) + b1[...])
    t = _relu(_mm(t, w2[...]) + b2[...])
    t = _relu(_mm(t, w3[...]) + b3[...])
    e40 = _mm(t, w4[...]) + b4[...]
    epad_o[...] = e40 @ se2[...]


def _node_o_body(h, a0, a1, wh, wa, b1, w2, b2, w3, b3, w4, b4, o):
    agg = a0[...] + a1[...]
    t = _relu(h[...] @ wh[...] + agg @ wa[...] + b1[...])
    t = _relu(t @ w2[...] + b2[...])
    t = _relu(t @ w3[...] + b3[...])
    o[...] = h[...] + (t @ w4[...] + b4[...])


def _node_final_body(h, a0, a1, wh, wa, b1, w2, b2, w3, b3, w4, b4,
                     bw1, bb1, bw2, bb2, bw3, bb3, bw4, bb4,
                     xw1, xb1, xw2, xb2, xw3, xb3, xw4, xb4,
                     hout_o, beta_o):
    agg = a0[...] + a1[...]
    t = _relu(h[...] @ wh[...] + agg @ wa[...] + b1[...])
    t = _relu(t @ w2[...] + b2[...])
    t = _relu(t @ w3[...] + b3[...])
    h2 = h[...] + (t @ w4[...] + b4[...])
    u = _relu(h2 @ bw1[...] + bb1[...])
    u = _relu(u @ bw2[...] + bb2[...])
    u = _relu(u @ bw3[...] + bb3[...])
    beta_o[...] = jax.nn.sigmoid(u @ bw4[...] + bb4[...])
    v = _relu(h2 @ xw1[...] + xb1[...])
    v = _relu(v @ xw2[...] + xb2[...])
    v = _relu(v @ xw3[...] + xb3[...])
    hout_o[...] = v @ xw4[...] + xb4[...]


def _tc_rows(body, n_out, out_shapes, blk, nblk, row_args, full_args,
             col_args=(), col_blk=0):
    in_specs = ([_rs(blk, a.shape[1]) for a in row_args]
                + [pl.BlockSpec((a.shape[0], col_blk), lambda i: (0, i))
                   for a in col_args]
                + [_fs(a) for a in full_args])
    row_args = list(row_args) + list(col_args)
    out_specs = tuple(_rs(blk, s[1]) for s in out_shapes)
    out_shape = tuple(jax.ShapeDtypeStruct(s, jnp.float32) for s in out_shapes)
    if n_out == 1:
        out_specs, out_shape = out_specs[0], out_shape[0]
    return pl.pallas_call(
        body, grid=(nblk,), in_specs=in_specs, out_specs=out_specs,
        out_shape=out_shape,
    )(*row_args, *full_args)


_N_PREP_IN = 30


def _prep_body(*refs):
    ins = refs[:_N_PREP_IN]
    outs = refs[_N_PREP_IN:]

    def bd(w, p, pad_rows=0):
        a, b = w.shape
        t = jnp.concatenate([w] * p, axis=0)
        t = jnp.concatenate([t] * p, axis=1)
        ri = lax.broadcasted_iota(jnp.int32, t.shape, 0) // a
        ci = lax.broadcasted_iota(jnp.int32, t.shape, 1) // b
        t = jnp.where(ri == ci, t, 0.0)
        if pad_rows:
            t = jnp.concatenate(
                [t, jnp.zeros((pad_rows, t.shape[1]), jnp.float32)], axis=0)
        return t

    def tl(b, p):
        return jnp.concatenate([b] * p, axis=1)

    (we1, be1, we2, be2, a1, ab1, a2, ab2, a3, ab3, a4, ab4,
     v1, vb1, v2, vb2, v3, vb3, v4, vb4,
     g1, gb1, g2, gb2, g3, gb3, g4, gb4, o1w, o2w) = [r[...] for r in ins]

    vals = [
        jnp.concatenate([jnp.transpose(we1), jnp.zeros((40, 4), jnp.float32)],
                        axis=1),
        jnp.transpose(be1), jnp.transpose(we2), jnp.transpose(be2),
        bd(a1[0:10], P1, 8), bd(a1[10:14], P1), tl(ab1, P1),
        bd(a2, P1), tl(ab2, P1), bd(a3, P1), tl(ab3, P1),
        bd(a4, P1), tl(ab4, P1),
        bd(v1, P1), tl(vb1, P1), bd(v2, P1), tl(vb2, P1),
        bd(v3, P1), tl(vb3, P1), bd(v4, P1), tl(vb4, P1),
        bd(g1, P2, 8), tl(gb1, P2), bd(g2, P2), tl(gb2, P2),
        bd(g3, P2), tl(gb3, P2), bd(g4, P2), tl(gb4, P2),
        o1w[0:5],
        jnp.concatenate([o1w[5:9], jnp.zeros((4, 40), jnp.float32)], axis=0),
        o2w[0:5],
        jnp.concatenate([o2w[5:10], jnp.zeros((3, 40), jnp.float32)], axis=0),
    ]
    for o, v in zip(outs, vals, strict=True):
        o[...] = v


def _prep_weights(ins):
    out_shapes = [
        (40, 8), (40, 1), (4, 40), (4, 1),
        (128, 480), (48, 480), (1, 480),
        (480, 480), (1, 480), (480, 480), (1, 480),
        (480, 48), (1, 48),
        (48, 480), (1, 480), (480, 480), (1, 480),
        (480, 480), (1, 480), (480, 12), (1, 12),
        (128, 320), (1, 320), (320, 320), (1, 320),
        (320, 320), (1, 320), (320, 40), (1, 40),
        (5, 40), (8, 40), (5, 40), (8, 40),
    ]
    return pl.pallas_call(
        _prep_body, grid=(1,),
        in_specs=[_fs(a) for a in ins],
        out_specs=tuple(pl.BlockSpec(s, lambda i: (0, 0)) for s in out_shapes),
        out_shape=tuple(jax.ShapeDtypeStruct(s, jnp.float32) for s in out_shapes),
    )(*ins)


def _make_gather(p, stride, with_eaw, ch):
    rpc = ch // p
    mlen = rpc * 128
    nch = EWK // ch
    ewlen = (ch // P1) * 128
    mesh = plsc.VectorSubcoreMesh(core_axis_name="c", subcore_axis_name="s",
                                  num_cores=NC, num_subcores=NS)
    scratch = [
        pltpu.VMEM((N_NODES * 5,), jnp.float32),
        pltpu.VMEM((ch,), jnp.int32), pltpu.VMEM((ch,), jnp.int32),
        pltpu.VMEM((ch,), jnp.int32), pltpu.VMEM((ch,), jnp.int32),
        pltpu.VMEM((mlen,), jnp.float32), pltpu.VMEM((mlen,), jnp.float32),
        pltpu.SemaphoreType.DMA, pltpu.SemaphoreType.DMA,
        pltpu.SemaphoreType.DMA, pltpu.SemaphoreType.DMA,
    ]
    if with_eaw:
        scratch += [pltpu.VMEM((ewlen,), jnp.float32),
                    pltpu.VMEM((ewlen,), jnp.float32)]

    def body(h_hbm, src_hbm, dst_hbm, *rest):
        if with_eaw:
            (eaw_hbm, out_hbm, tab_v, si0, si1, di0, di1, m0, m1,
             sin0, sin1, so0, so1, ew0, ew1) = rest
            ew = [ew0, ew1]
        else:
            (out_hbm, tab_v, si0, si1, di0, di1, m0, m1,
             sin0, sin1, so0, so1) = rest
            ew = None
        si, di, m = [si0, si1], [di0, di1], [m0, m1]
        sin, so = [sin0, sin1], [so0, so1]
        wid = lax.axis_index("s") * NC + lax.axis_index("c")
        lanes = lax.iota(jnp.int32, 16)
        zero16 = jnp.zeros((16,), jnp.float32)

        def in_copies(kk, s):
            base = wid * EWK + kk * ch
            yield src_hbm.at[pl.ds(base, ch)], si[s], sin[s]
            yield dst_hbm.at[pl.ds(base, ch)], di[s], sin[s]
            if with_eaw:
                yield eaw_hbm.at[pl.ds((base // P1) * 128, ewlen)], ew[s], sin[s]

        def out_copy(kk, s):
            base = wid * EWK + kk * ch
            return m[s], out_hbm.at[pl.ds((base // p) * 128, mlen)], so[s]

        for t in in_copies(0, 0):
            pltpu.async_copy(*t)
        pltpu.sync_copy(h_hbm, tab_v)

        zpad = 128 - p * stride

        def zfill(mv):
            @plsc.parallel_loop(0, rpc * zpad // 16, unroll=8)
            def _zf(i):
                v = i * 16 + lanes
                plsc.store_scatter(mv, [(v // zpad) * 128 + p * stride + (v % zpad)],
                                   zero16)

        zfill(m0)
        zfill(m1)

        for kk in range(nch):
            s = kk % 2
            if kk + 1 < nch:
                for t in in_copies(kk + 1, 1 - s):
                    pltpu.async_copy(*t)
            for t in in_copies(kk, s):
                pltpu.make_async_copy(*t).wait()
            if kk >= 2:
                pltpu.make_async_copy(*out_copy(kk - 2, s)).wait()

            @plsc.parallel_loop(0, ch // 16, unroll=4)
            def _grp(i):
                q = i * 16 + lanes
                d5 = di[s][pl.ds(i * 16, 16)] * 5
                s5 = si[s][pl.ds(i * 16, 16)] * 5
                moff = (q // p) * 128 + (q % p) * stride
                for c in range(5):
                    vd = plsc.load_gather(tab_v, [d5 + c])
                    plsc.store_scatter(m[s], [moff + c], vd)
                    vs = plsc.load_gather(tab_v, [s5 + c])
                    plsc.store_scatter(m[s], [moff + 5 + c], vs)
                if with_eaw:
                    eoff = (q // P1) * 128 + (q % P1) * 10
                    for c in range(5):
                        ve = plsc.load_gather(ew[s], [eoff + c])
                        plsc.store_scatter(m[s], [moff + 10 + c], ve)
            pltpu.async_copy(*out_copy(kk, s))

        for kk in range(max(nch - 2, 0), nch):
            pltpu.make_async_copy(*out_copy(kk, kk % 2)).wait()

    n_rows = EP // p
    return functools.partial(
        pl.kernel, mesh=mesh,
        compiler_params=pltpu.CompilerParams(needs_layout_passes=False),
        out_type=jax.ShapeDtypeStruct((n_rows * 128,), jnp.float32),
        scratch_types=scratch,
    )(body)


def _make_scatter(p, stride):
    rpc = CHS // p
    elen = rpc * 128
    nch = EWK // CHS
    mesh = plsc.VectorSubcoreMesh(core_axis_name="c", subcore_axis_name="s",
                                  num_cores=NC, num_subcores=NS)

    @functools.partial(
        pl.kernel, mesh=mesh,
        compiler_params=pltpu.CompilerParams(needs_layout_passes=False,
                                             use_tc_tiling_on_sc=False),
        out_type=jax.ShapeDtypeStruct((NC, NACC, 8), jnp.float32),
        scratch_types=[
            pltpu.VMEM((elen,), jnp.float32), pltpu.VMEM((elen,), jnp.float32),
            pltpu.VMEM((NBAT, SB), jnp.int32), pltpu.VMEM((NBAT, SB), jnp.int32),
            pltpu.VMEM((CHS, 8), jnp.float32),
            pltpu.VMEM((NROW_T, 8), jnp.float32),
            pltpu.VMEM_SHARED((NACC, 8), jnp.float32),
            pltpu.SemaphoreType.DMA, pltpu.SemaphoreType.DMA,
            pltpu.SemaphoreType.DMA,
        ],
    )
    def body(e_hbm, d_hbm, out_hbm, e0, e1, i0, i1, s_v, zo_v, acc_sh,
             sin0, sin1, sadd):
        cid = lax.axis_index("c")
        sid = lax.axis_index("s")
        wid = sid * NC + cid
        lanes = lax.iota(jnp.int32, 16)
        zero16 = jnp.zeros((16,), jnp.float32)
        e, iv, sin = [e0, e1], [i0, i1], [sin0, sin1]

        def in_copies(kk, s):
            base = wid * EWK + kk * CHS
            yield e_hbm.at[pl.ds((base // p) * 128, elen)], e[s], sin[s]
            yield d_hbm.at[pl.ds(wid * (nch * NBAT) + kk * NBAT, NBAT)], iv[s], sin[s]

        for t in in_copies(0, 0):
            pltpu.async_copy(*t)

        @plsc.parallel_loop(0, NROW_T * 8 // 16, unroll=8)
        def _zf(i):
            v = i * 16 + lanes
            plsc.store_scatter(zo_v, [v // 8, v % 8], zero16)
        pltpu.sync_copy(zo_v, acc_sh.at[pl.ds(sid * NROW_T, NROW_T)])
        plsc.subcore_barrier()

        for kk in range(nch):
            s = kk % 2
            if kk + 1 < nch:
                for t in in_copies(kk + 1, 1 - s):
                    pltpu.async_copy(*t)
            for t in in_copies(kk, s):
                pltpu.make_async_copy(*t).wait()

            def batch(b, _):
                @plsc.parallel_loop(0, SB // 16, unroll=7)
                def _fill(g):
                    t0 = b * SB + g * 16
                    q = t0 + lanes
                    eoff = (q // p) * 128 + (q % p) * stride
                    for c in range(8):
                        v = plsc.load_gather(e[s], [eoff + c])
                        plsc.store_scatter(
                            s_v, [q, jnp.full((16,), c, jnp.int32)], v)
                pltpu.async_copy(s_v.at[pl.ds(b * SB, SB)],
                                 acc_sh.at[iv[s].at[b]], sadd, add=True)
                return 0

            lax.fori_loop(0, NBAT, batch, 0)

            def drain(b, _):
                pltpu.make_async_copy(s_v.at[pl.ds(b * SB, SB)],
                                      acc_sh.at[iv[s].at[b]], sadd).wait()
                return 0

            lax.fori_loop(0, NBAT, drain, 0)

        plsc.subcore_barrier()
        pltpu.sync_copy(acc_sh.at[pl.ds(sid * NROW_T, NROW_T)], zo_v)
        pltpu.sync_copy(zo_v, out_hbm.at[cid, pl.ds(sid * NROW_T, NROW_T)])

    return body


@functools.cache
def _sc_kernels():
    return (_make_gather(P1, 10, False, CH1), _make_gather(P2, 15, True, CH2),
            _make_scatter(P1, 10), _make_scatter(P2, 16))


def kernel(x, edge_index, edge_attr, params):
    src = edge_index[0]
    dst = edge_index[1]

    def permute(a):
        return a.reshape(NBLK1, P1, EBLK1).transpose(0, 2, 1).reshape(-1)

    zpad_i = jnp.zeros((PAD_E,), jnp.int32)
    src_p = permute(jnp.concatenate([src, zpad_i]))
    dst_g = permute(jnp.concatenate([dst, zpad_i]))
    dst_s = permute(jnp.concatenate(
        [dst, 10000 + (jnp.arange(PAD_E, dtype=jnp.int32) % 240)])).reshape(-1, SB)
    eat8 = jnp.pad(edge_attr.T, ((0, 4), (0, PAD_E)))

    pn = params['node_encoder']
    pe = params['edge_encoder']
    ec = params['ec_layers'][0]
    hc = params['hc_layers'][0]
    pw, pb, px = params['W'], params['B'], params['X']

    def b2(v):
        return v[None, :]

    g1, g2, s1, s2 = _sc_kernels()

    r1 = ec['R1']
    r2 = hc['R1']
    o1 = ec['O']
    o2 = hc['O']
    prep_in = [
        pe[0][0], b2(pe[0][1]), pe[1][0], b2(pe[1][1]),
        r1[0][0], b2(r1[0][1]), r1[1][0], b2(r1[1][1]),
        r1[2][0], b2(r1[2][1]), r1[3][0], b2(r1[3][1]),
        pw[0][0], b2(pw[0][1]), pw[1][0], b2(pw[1][1]),
        pw[2][0], b2(pw[2][1]), pw[3][0], b2(pw[3][1]),
        r2[0][0], b2(r2[0][1]), r2[1][0], b2(r2[1][1]),
        r2[2][0], b2(r2[2][1]), r2[3][0], b2(r2[3][1]),
        o1[0][0], o2[0][0],
    ]
    pw_out = _prep_weights(prep_in)
    (we1b, be1b, we2b, be2b, w1m, w1e, b1b, w2b, b2b, w3b, b3b, w4b, b4b,
     v1b, c1b, v2b, c2b, v3b, c3b, v4b, c4b,
     w1m2, bh1, wh2, bh2, wh3, bh3, wh4, bh4,
     woh1, woa1, woh2, woa2) = pw_out

    h0 = _tc_rows(_mlp2_body, 1, [(N_NODES, 5)], NODE_BLK, NBLK_N,
                  [x], [pn[0][0], b2(pn[0][1]), pn[1][0], b2(pn[1][1])])

    m1 = g1(h0.reshape(-1), src_p, dst_g).reshape(R1R, 128)

    epad1, eaw, w12 = _tc_rows(
        _ec_edge_body, 3, [(R1R, 128), (R1R, 128), (R1R, 12)], EBLK1, NBLK1,
        [m1],
        [we1b, be1b, we2b, be2b, w1m, w1e, b1b, w2b, b2b, w3b, b3b, w4b, b4b,
         v1b, c1b, v2b, c2b, v3b, c3b, v4b, c4b,
         S_E1, S_W1, S_EA1],
        col_args=[eat8], col_blk=CB1)

    agg1 = s1(epad1.reshape(-1), dst_s)[:, :N_NODES]

    h1 = _tc_rows(_node_o_body, 1, [(N_NODES, 5)], NODE_BLK, NBLK_N,
                  [h0, agg1[0], agg1[1]],
                  [woh1, woa1, b2(o1[0][1]),
                   o1[1][0], b2(o1[1][1]), o1[2][0], b2(o1[2][1]),
                   o1[3][0], b2(o1[3][1])])

    m2 = g2(h1.reshape(-1), src_p, dst_g, eaw.reshape(-1)).reshape(R2R, 128)

    epad2 = _tc_rows(
        _hc_edge_body, 1, [(R2R, 128)], EBLK2, NBLK2,
        [m2],
        [w1m2, bh1, wh2, bh2, wh3, bh3, wh4, bh4, S_E2])

    agg2 = s2(epad2.reshape(-1), dst_s)[:, :N_NODES]
    h_out, beta = _tc_rows(
        _node_final_body, 2, [(N_NODES, 2), (N_NODES, 1)], NODE_BLK, NBLK_N,
        [h1, agg2[0], agg2[1]],
        [woh2, woa2, b2(o2[0][1]),
         o2[1][0], b2(o2[1][1]), o2[2][0], b2(o2[2][1]),
         o2[3][0], b2(o2[3][1]),
         pb[0][0], b2(pb[0][1]), pb[1][0], b2(pb[1][1]),
         pb[2][0], b2(pb[2][1]), pb[3][0], b2(pb[3][1]),
         px[0][0], b2(px[0][1]), px[1][0], b2(px[1][1]),
         px[2][0], b2(px[2][1]), px[3][0], b2(px[3][1])])

    ew = (w12.reshape(NBLK1, EBLK1, P1).transpose(0, 2, 1)
          .reshape(-1)[:N_EDGES].reshape(N_EDGES, 1))
    return (ew, h_out, beta)


# speedup vs baseline: 9.3118x; 1.0201x over previous
import functools

import numpy as np
import jax
import jax.numpy as jnp
from jax import lax
from jax.experimental import pallas as pl
from jax.experimental.pallas import tpu as pltpu
from jax.experimental.pallas import tpu_sc as plsc

N_NODES = 10000
N_EDGES = 320000
EP = 322560
PAD_E = EP - N_EDGES
P1 = 12
P2 = 8
R1R = EP // P1
R2R = EP // P2
NC, NS = 2, 16
NW = NC * NS
EWK = EP // NW
CH1 = 2016
CH2 = 1008
CHS = 2016
SB = 112
NBAT = CHS // SB
NACC = 10240
NROW_T = NACC // NS

EBLK1 = 1120
NBLK1 = R1R // EBLK1
CB1 = EBLK1 * P1
EBLK2 = 2016
NBLK2 = R2R // EBLK2
NBLK_N = 5
NODE_BLK = N_NODES // NBLK_N


def _np_sel(n_in, n_out, pairs):
    s = np.zeros((n_in, n_out), np.float32)
    for i, j in pairs:
        s[i, j] = 1.0
    return s

S_E1 = _np_sel(48, 128, [(4 * p + c, 10 * p + c) for p in range(P1) for c in range(4)])
S_W1 = _np_sel(12, 128, [(p, 10 * p) for p in range(P1)])
S_EA1 = _np_sel(48, 128, [(4 * p + c, 10 * p + 1 + c) for p in range(P1) for c in range(4)])
S_E2 = _np_sel(40, 128, [(5 * p + c, 16 * p + c) for p in range(P2) for c in range(5)])


def _relu(v):
    return jnp.maximum(v, 0.0)


def _mm(x, w):
    return jnp.dot(x.astype(jnp.bfloat16), w.astype(jnp.bfloat16),
                   preferred_element_type=jnp.float32)


def _bd(w, p):
    return jnp.kron(jnp.eye(p, dtype=jnp.float32), w)


def _bdb(b, p):
    return jnp.tile(b, p)[None, :]


def _rs(blk, d):
    return pl.BlockSpec((blk, d), lambda i: (i, 0))


def _fs(a):
    return pl.BlockSpec(a.shape, lambda i: (0,) * a.ndim)


def _mlp2_body(x, w1, b1, w2, b2, o):
    t = _relu(x[...] @ w1[...] + b1[...])
    o[...] = _relu(t @ w2[...] + b2[...])


def _ec_edge_body(m, eat, we1t, be1t, we2t, be2t, w1m, w1e, b1, w2, b2, w3, b3,
                  w4, b4, v1, c1, v2, c2, v3, c3, v4, c4,
                  se1, sw1, sea1, epad_o, eaw_o, w12_o):
    ht = _relu(we1t[...] @ eat[...] + be1t[...])
    e0t = _relu(we2t[...] @ ht + be2t[...])
    ea0 = jnp.concatenate(
        [jnp.transpose(e0t[:, p * EBLK1:(p + 1) * EBLK1]) for p in range(P1)],
        axis=1)
    t = _relu(_mm(m[...], w1m[...]) + ea0 @ w1e[...] + b1[...])
    t = _relu(_mm(t, w2[...]) + b2[...])
    t = _relu(_mm(t, w3[...]) + b3[...])
    e48 = _mm(t, w4[...]) + b4[...]
    ea1 = ea0 + e48
    epad_o[...] = e48 @ se1[...]
    u = _relu(ea1 @ v1[...] + c1[...])
    u = _relu(_mm(u, v2[...]) + c2[...])
    u = _relu(_mm(u, v3[...]) + c3[...])
    w12 = jax.nn.sigmoid(_mm(u, v4[...]) + c4[...])
    w12_o[...] = w12
    eaw_o[...] = w12 @ sw1[...] + ea1 @ sea1[...]


def _hc_edge_body(m, w1m, b1, w2, b2, w3, b3, w4, b4, se2, epad_o):
    t = _relu(_mm(m[...], w1m[...]) + b1[...])
    t = _relu(_mm(t, w2[...]) + b2[...])
    t = _relu(_mm(t, w3[...]) + b3[...])
    e40 = _mm(t, w4[...]) + b4[...]
    epad_o[...] = e40 @ se2[...]


def _node_o_body(h, a0, a1, wh, wa, b1, w2, b2, w3, b3, w4, b4, o):
    agg = a0[...] + a1[...]
    t = _relu(h[...] @ wh[...] + agg @ wa[...] + b1[...])
    t = _relu(t @ w2[...] + b2[...])
    t = _relu(t @ w3[...] + b3[...])
    o[...] = h[...] + (t @ w4[...] + b4[...])


def _node_final_body(h, a0, a1, wh, wa, b1, w2, b2, w3, b3, w4, b4,
                     bw1, bb1, bw2, bb2, bw3, bb3, bw4, bb4,
                     xw1, xb1, xw2, xb2, xw3, xb3, xw4, xb4,
                     hout_o, beta_o):
    agg = a0[...] + a1[...]
    t = _relu(h[...] @ wh[...] + agg @ wa[...] + b1[...])
    t = _relu(t @ w2[...] + b2[...])
    t = _relu(t @ w3[...] + b3[...])
    h2 = h[...] + (t @ w4[...] + b4[...])
    u = _relu(h2 @ bw1[...] + bb1[...])
    u = _relu(u @ bw2[...] + bb2[...])
    u = _relu(u @ bw3[...] + bb3[...])
    beta_o[...] = jax.nn.sigmoid(u @ bw4[...] + bb4[...])
    v = _relu(h2 @ xw1[...] + xb1[...])
    v = _relu(v @ xw2[...] + xb2[...])
    v = _relu(v @ xw3[...] + xb3[...])
    hout_o[...] = v @ xw4[...] + xb4[...]


def _tc_rows(body, n_out, out_shapes, blk, nblk, row_args, full_args,
             col_args=(), col_blk=0):
    in_specs = ([_rs(blk, a.shape[1]) for a in row_args]
                + [pl.BlockSpec((a.shape[0], col_blk), lambda i: (0, i))
                   for a in col_args]
                + [_fs(a) for a in full_args])
    row_args = list(row_args) + list(col_args)
    out_specs = tuple(_rs(blk, s[1]) for s in out_shapes)
    out_shape = tuple(jax.ShapeDtypeStruct(s, jnp.float32) for s in out_shapes)
    if n_out == 1:
        out_specs, out_shape = out_specs[0], out_shape[0]
    return pl.pallas_call(
        body, grid=(nblk,), in_specs=in_specs, out_specs=out_specs,
        out_shape=out_shape,
    )(*row_args, *full_args)


_N_PREP_IN = 30


def _prep_body(*refs):
    ins = refs[:_N_PREP_IN]
    outs = refs[_N_PREP_IN:]

    def bd(w, p, pad_rows=0):
        a, b = w.shape
        t = jnp.concatenate([w] * p, axis=0)
        t = jnp.concatenate([t] * p, axis=1)
        ri = lax.broadcasted_iota(jnp.int32, t.shape, 0) // a
        ci = lax.broadcasted_iota(jnp.int32, t.shape, 1) // b
        t = jnp.where(ri == ci, t, 0.0)
        if pad_rows:
            t = jnp.concatenate(
                [t, jnp.zeros((pad_rows, t.shape[1]), jnp.float32)], axis=0)
        return t

    def tl(b, p):
        return jnp.concatenate([b] * p, axis=1)

    (we1, be1, we2, be2, a1, ab1, a2, ab2, a3, ab3, a4, ab4,
     v1, vb1, v2, vb2, v3, vb3, v4, vb4,
     g1, gb1, g2, gb2, g3, gb3, g4, gb4, o1w, o2w) = [r[...] for r in ins]

    vals = [
        jnp.concatenate([jnp.transpose(we1), jnp.zeros((40, 4), jnp.float32)],
                        axis=1),
        jnp.transpose(be1), jnp.transpose(we2), jnp.transpose(be2),
        bd(a1[0:10], P1, 8), bd(a1[10:14], P1), tl(ab1, P1),
        bd(a2, P1), tl(ab2, P1), bd(a3, P1), tl(ab3, P1),
        bd(a4, P1), tl(ab4, P1),
        bd(v1, P1), tl(vb1, P1), bd(v2, P1), tl(vb2, P1),
        bd(v3, P1), tl(vb3, P1), bd(v4, P1), tl(vb4, P1),
        bd(g1, P2, 8), tl(gb1, P2), bd(g2, P2), tl(gb2, P2),
        bd(g3, P2), tl(gb3, P2), bd(g4, P2), tl(gb4, P2),
        o1w[0:5],
        jnp.concatenate([o1w[5:9], jnp.zeros((4, 40), jnp.float32)], axis=0),
        o2w[0:5],
        jnp.concatenate([o2w[5:10], jnp.zeros((3, 40), jnp.float32)], axis=0),
    ]
    for o, v in zip(outs, vals, strict=True):
        o[...] = v


def _prep_weights(ins):
    out_shapes = [
        (40, 8), (40, 1), (4, 40), (4, 1),
        (128, 480), (48, 480), (1, 480),
        (480, 480), (1, 480), (480, 480), (1, 480),
        (480, 48), (1, 48),
        (48, 480), (1, 480), (480, 480), (1, 480),
        (480, 480), (1, 480), (480, 12), (1, 12),
        (128, 320), (1, 320), (320, 320), (1, 320),
        (320, 320), (1, 320), (320, 40), (1, 40),
        (5, 40), (8, 40), (5, 40), (8, 40),
    ]
    return pl.pallas_call(
        _prep_body, grid=(1,),
        in_specs=[_fs(a) for a in ins],
        out_specs=tuple(pl.BlockSpec(s, lambda i: (0, 0)) for s in out_shapes),
        out_shape=tuple(jax.ShapeDtypeStruct(s, jnp.float32) for s in out_shapes),
    )(*ins)


def _make_gather(p, stride, with_eaw, ch):
    rpc = ch // p
    mlen = rpc * 128
    nch = EWK // ch
    ewlen = (ch // P1) * 128
    mesh = plsc.VectorSubcoreMesh(core_axis_name="c", subcore_axis_name="s",
                                  num_cores=NC, num_subcores=NS)
    scratch = [
        pltpu.VMEM((N_NODES * 5,), jnp.float32),
        pltpu.VMEM((ch,), jnp.int32), pltpu.VMEM((ch,), jnp.int32),
        pltpu.VMEM((ch,), jnp.int32), pltpu.VMEM((ch,), jnp.int32),
        pltpu.VMEM((mlen,), jnp.float32), pltpu.VMEM((mlen,), jnp.float32),
        pltpu.SemaphoreType.DMA, pltpu.SemaphoreType.DMA,
        pltpu.SemaphoreType.DMA, pltpu.SemaphoreType.DMA,
    ]
    if with_eaw:
        scratch += [pltpu.VMEM((ewlen,), jnp.float32),
                    pltpu.VMEM((ewlen,), jnp.float32)]

    def body(h_hbm, src_hbm, dst_hbm, *rest):
        if with_eaw:
            (eaw_hbm, out_hbm, tab_v, si0, si1, di0, di1, m0, m1,
             sin0, sin1, so0, so1, ew0, ew1) = rest
            ew = [ew0, ew1]
        else:
            (out_hbm, tab_v, si0, si1, di0, di1, m0, m1,
             sin0, sin1, so0, so1) = rest
            ew = None
        si, di, m = [si0, si1], [di0, di1], [m0, m1]
        sin, so = [sin0, sin1], [so0, so1]
        wid = lax.axis_index("s") * NC + lax.axis_index("c")
        lanes = lax.iota(jnp.int32, 16)
        zero16 = jnp.zeros((16,), jnp.float32)

        def in_copies(kk, s):
            base = wid * EWK + kk * ch
            yield src_hbm.at[pl.ds(base, ch)], si[s], sin[s]
            yield dst_hbm.at[pl.ds(base, ch)], di[s], sin[s]
            if with_eaw:
                yield eaw_hbm.at[pl.ds((base // P1) * 128, ewlen)], ew[s], sin[s]

        def out_copy(kk, s):
            base = wid * EWK + kk * ch
            return m[s], out_hbm.at[pl.ds((base // p) * 128, mlen)], so[s]

        for t in in_copies(0, 0):
            pltpu.async_copy(*t)
        pltpu.sync_copy(h_hbm, tab_v)

        zpad = 128 - p * stride

        def zfill(mv):
            @plsc.parallel_loop(0, rpc * zpad // 16, unroll=8)
            def _zf(i):
                v = i * 16 + lanes
                plsc.store_scatter(mv, [(v // zpad) * 128 + p * stride + (v % zpad)],
                                   zero16)

        zfill(m0)
        zfill(m1)

        for kk in range(nch):
            s = kk % 2
            if kk + 1 < nch:
                for t in in_copies(kk + 1, 1 - s):
                    pltpu.async_copy(*t)
            for t in in_copies(kk, s):
                pltpu.make_async_copy(*t).wait()
            if kk >= 2:
                pltpu.make_async_copy(*out_copy(kk - 2, s)).wait()

            @plsc.parallel_loop(0, ch // 16, unroll=4)
            def _grp(i):
                q = i * 16 + lanes
                d5 = di[s][pl.ds(i * 16, 16)] * 5
                s5 = si[s][pl.ds(i * 16, 16)] * 5
                moff = (q // p) * 128 + (q % p) * stride
                for c in range(5):
                    vd = plsc.load_gather(tab_v, [d5 + c])
                    plsc.store_scatter(m[s], [moff + c], vd)
                    vs = plsc.load_gather(tab_v, [s5 + c])
                    plsc.store_scatter(m[s], [moff + 5 + c], vs)
                if with_eaw:
                    eoff = (q // P1) * 128 + (q % P1) * 10
                    for c in range(5):
                        ve = plsc.load_gather(ew[s], [eoff + c])
                        plsc.store_scatter(m[s], [moff + 10 + c], ve)
            pltpu.async_copy(*out_copy(kk, s))

        for kk in range(max(nch - 2, 0), nch):
            pltpu.make_async_copy(*out_copy(kk, kk % 2)).wait()

    n_rows = EP // p
    return functools.partial(
        pl.kernel, mesh=mesh,
        compiler_params=pltpu.CompilerParams(needs_layout_passes=False),
        out_type=jax.ShapeDtypeStruct((n_rows * 128,), jnp.float32),
        scratch_types=scratch,
    )(body)


def _make_scatter(p, stride):
    rpc = CHS // p
    elen = rpc * 128
    nch = EWK // CHS
    mesh = plsc.VectorSubcoreMesh(core_axis_name="c", subcore_axis_name="s",
                                  num_cores=NC, num_subcores=NS)

    @functools.partial(
        pl.kernel, mesh=mesh,
        compiler_params=pltpu.CompilerParams(needs_layout_passes=False,
                                             use_tc_tiling_on_sc=False),
        out_type=jax.ShapeDtypeStruct((NC, NACC, 8), jnp.float32),
        scratch_types=[
            pltpu.VMEM((elen,), jnp.float32), pltpu.VMEM((elen,), jnp.float32),
            pltpu.VMEM((NBAT, SB), jnp.int32), pltpu.VMEM((NBAT, SB), jnp.int32),
            pltpu.VMEM((CHS, 8), jnp.float32),
            pltpu.VMEM((NROW_T, 8), jnp.float32),
            pltpu.VMEM_SHARED((NACC, 8), jnp.float32),
            pltpu.SemaphoreType.DMA, pltpu.SemaphoreType.DMA,
            pltpu.SemaphoreType.DMA,
        ],
    )
    def body(e_hbm, d_hbm, out_hbm, e0, e1, i0, i1, s_v, zo_v, acc_sh,
             sin0, sin1, sadd):
        cid = lax.axis_index("c")
        sid = lax.axis_index("s")
        wid = sid * NC + cid
        lanes = lax.iota(jnp.int32, 16)
        zero16 = jnp.zeros((16,), jnp.float32)
        e, iv, sin = [e0, e1], [i0, i1], [sin0, sin1]

        def in_copies(kk, s):
            base = wid * EWK + kk * CHS
            yield e_hbm.at[pl.ds((base // p) * 128, elen)], e[s], sin[s]
            yield d_hbm.at[pl.ds(wid * (nch * NBAT) + kk * NBAT, NBAT)], iv[s], sin[s]

        for t in in_copies(0, 0):
            pltpu.async_copy(*t)

        @plsc.parallel_loop(0, NROW_T * 8 // 16, unroll=8)
        def _zf(i):
            v = i * 16 + lanes
            plsc.store_scatter(zo_v, [v // 8, v % 8], zero16)
        pltpu.sync_copy(zo_v, acc_sh.at[pl.ds(sid * NROW_T, NROW_T)])
        plsc.subcore_barrier()

        for kk in range(nch):
            s = kk % 2
            if kk + 1 < nch:
                for t in in_copies(kk + 1, 1 - s):
                    pltpu.async_copy(*t)
            for t in in_copies(kk, s):
                pltpu.make_async_copy(*t).wait()

            def batch(b, _):
                @plsc.parallel_loop(0, SB // 16, unroll=7)
                def _fill(g):
                    t0 = b * SB + g * 16
                    q = t0 + lanes
                    eoff = (q // p) * 128 + (q % p) * stride
                    for c in range(8):
                        v = plsc.load_gather(e[s], [eoff + c])
                        plsc.store_scatter(
                            s_v, [q, jnp.full((16,), c, jnp.int32)], v)
                pltpu.async_copy(s_v.at[pl.ds(b * SB, SB)],
                                 acc_sh.at[iv[s].at[b]], sadd, add=True)
                return 0

            lax.fori_loop(0, NBAT, batch, 0)

            def drain(b, _):
                pltpu.make_async_copy(s_v.at[pl.ds(b * SB, SB)],
                                      acc_sh.at[iv[s].at[b]], sadd).wait()
                return 0

            lax.fori_loop(0, NBAT, drain, 0)

        plsc.subcore_barrier()
        pltpu.sync_copy(acc_sh.at[pl.ds(sid * NROW_T, NROW_T)], zo_v)
        pltpu.sync_copy(zo_v, out_hbm.at[cid, pl.ds(sid * NROW_T, NROW_T)])

    return body


@functools.cache
def _sc_kernels():
    return (_make_gather(P1, 10, False, CH1), _make_gather(P2, 15, True, CH2),
            _make_scatter(P1, 10), _make_scatter(P2, 16))


def kernel(x, edge_index, edge_attr, params):
    src = edge_index[0]
    dst = edge_index[1]

    def permute(a):
        return a.reshape(NBLK1, P1, EBLK1).transpose(0, 2, 1).reshape(-1)

    zpad_i = jnp.zeros((PAD_E,), jnp.int32)
    src_p = permute(jnp.concatenate([src, zpad_i]))
    dst_g = permute(jnp.concatenate([dst, zpad_i]))
    dst_s = permute(jnp.concatenate(
        [dst, 10000 + (jnp.arange(PAD_E, dtype=jnp.int32) % 240)])).reshape(-1, SB)
    eat8 = jnp.pad(edge_attr.T, ((0, 4), (0, PAD_E)))

    pn = params['node_encoder']
    pe = params['edge_encoder']
    ec = params['ec_layers'][0]
    hc = params['hc_layers'][0]
    pw, pb, px = params['W'], params['B'], params['X']

    def b2(v):
        return v[None, :]

    g1, g2, s1, s2 = _sc_kernels()

    r1 = ec['R1']
    r2 = hc['R1']
    o1 = ec['O']
    o2 = hc['O']
    prep_in = [
        pe[0][0], b2(pe[0][1]), pe[1][0], b2(pe[1][1]),
        r1[0][0], b2(r1[0][1]), r1[1][0], b2(r1[1][1]),
        r1[2][0], b2(r1[2][1]), r1[3][0], b2(r1[3][1]),
        pw[0][0], b2(pw[0][1]), pw[1][0], b2(pw[1][1]),
        pw[2][0], b2(pw[2][1]), pw[3][0], b2(pw[3][1]),
        r2[0][0], b2(r2[0][1]), r2[1][0], b2(r2[1][1]),
        r2[2][0], b2(r2[2][1]), r2[3][0], b2(r2[3][1]),
        o1[0][0], o2[0][0],
    ]
    pw_out = _prep_weights(prep_in)
    (we1b, be1b, we2b, be2b, w1m, w1e, b1b, w2b, b2b, w3b, b3b, w4b, b4b,
     v1b, c1b, v2b, c2b, v3b, c3b, v4b, c4b,
     w1m2, bh1, wh2, bh2, wh3, bh3, wh4, bh4,
     woh1, woa1, woh2, woa2) = pw_out

    h0 = _tc_rows(_mlp2_body, 1, [(N_NODES, 5)], NODE_BLK, NBLK_N,
                  [x], [pn[0][0], b2(pn[0][1]), pn[1][0], b2(pn[1][1])])

    m1 = g1(h0.reshape(-1), src_p, dst_g).reshape(R1R, 128)

    epad1, eaw, w12 = _tc_rows(
        _ec_edge_body, 3, [(R1R, 128), (R1R, 128), (R1R, 12)], EBLK1, NBLK1,
        [m1],
        [we1b, be1b, we2b, be2b, w1m, w1e, b1b, w2b, b2b, w3b, b3b, w4b, b4b,
         v1b, c1b, v2b, c2b, v3b, c3b, v4b, c4b,
         S_E1, S_W1, S_EA1],
        col_args=[eat8], col_blk=CB1)

    agg1 = s1(epad1.reshape(-1), dst_s)[:, :N_NODES]

    h1 = _tc_rows(_node_o_body, 1, [(N_NODES, 5)], NODE_BLK, NBLK_N,
                  [h0, agg1[0], agg1[1]],
                  [woh1, woa1, b2(o1[0][1]),
                   o1[1][0], b2(o1[1][1]), o1[2][0], b2(o1[2][1]),
                   o1[3][0], b2(o1[3][1])])

    m2 = g2(h1.reshape(-1), src_p, dst_g, eaw.reshape(-1)).reshape(R2R, 128)

    epad2 = _tc_rows(
        _hc_edge_body, 1, [(R2R, 128)], EBLK2, NBLK2,
        [m2],
        [w1m2, bh1, wh2, bh2, wh3, bh3, wh4, bh4, S_E2])

    agg2 = s2(epad2.reshape(-1), dst_s)[:, :N_NODES]
    h_out, beta = _tc_rows(
        _node_final_body, 2, [(N_NODES, 2), (N_NODES, 1)], NODE_BLK, NBLK_N,
        [h1, agg2[0], agg2[1]],
        [woh2, woa2, b2(o2[0][1]),
         o2[1][0], b2(o2[1][1]), o2[2][0], b2(o2[2][1]),
         o2[3][0], b2(o2[3][1]),
         pb[0][0], b2(pb[0][1]), pb[1][0], b2(pb[1][1]),
         pb[2][0], b2(pb[2][1]), pb[3][0], b2(pb[3][1]),
         px[0][0], b2(px[0][1]), px[1][0], b2(px[1][1]),
         px[2][0], b2(px[2][1]), px[3][0], b2(px[3][1])])

    ew = (w12.reshape(NBLK1, EBLK1, P1).transpose(0, 2, 1)
          .reshape(-1)[:N_EDGES].reshape(N_EDGES, 1))
    return (ew, h_out, beta)
